# baseline (speedup 1.0000x reference)
import functools

import jax
import jax.numpy as jnp
import numpy as np
from jax import lax
from jax.experimental import pallas as pl
from jax.experimental.pallas import tpu as pltpu
from jax.experimental.pallas import tpu_sc as plsc

N = 10000
E = 320000
D = 128
NC = 1000
CD = 8
HEADS = 8
HID = 16
OUT = 64

NWORKERS = 32
KCH = 80
EPW = E // NWORKERS
NCHUNK = EPW // KCH
IB = 25
W1COLS = HEADS * HID
GW1 = 16 + W1COLS
GW2 = 16 + OUT


def _comm_gather_kernel(n_pad):
  per_w = n_pad // NWORKERS
  steps = per_w // KCH
  mesh = plsc.VectorSubcoreMesh(core_axis_name="c", subcore_axis_name="s")

  @functools.partial(
      pl.kernel,
      mesh=mesh,
      out_type=jax.ShapeDtypeStruct((n_pad, 16), jnp.float32),
      compiler_params=pltpu.CompilerParams(use_tc_tiling_on_sc=False),
      scratch_types=[
          pltpu.VMEM((steps, KCH), jnp.int32),
          pltpu.VMEM((KCH, 16), jnp.float32),
          pltpu.SemaphoreType.DMA,
      ],
  )
  def k(ids_hbm, ctab_hbm, out_hbm, idxb, rows, sem):
    cid = lax.axis_index("c")
    sid = lax.axis_index("s")
    wid = cid * 16 + sid
    pltpu.sync_copy(ids_hbm.at[pl.ds(wid * steps, steps)], idxb)
    for kk in range(steps):
      pltpu.async_copy(ctab_hbm.at[idxb.at[kk]], rows, sem).wait()
      pltpu.sync_copy(rows, out_hbm.at[pl.ds(wid * per_w + kk * KCH, KCH)])

  return k


def _lane_bcast(v, lane):
  idx = jnp.full((16, 1), lane, jnp.int32)
  return lax.gather(
      v, idx,
      lax.GatherDimensionNumbers(
          offset_dims=(), collapsed_slice_dims=(0,), start_index_map=(0,)),
      (1,),
      mode=lax.GatherScatterMode.PROMISE_IN_BOUNDS)


def _edge_kernel(width, heads):
  rows_per_tile = N // 16
  mesh = plsc.VectorSubcoreMesh(core_axis_name="c", subcore_axis_name="s")

  @functools.partial(
      pl.kernel,
      mesh=mesh,
      out_type=jax.ShapeDtypeStruct((2, N, width), jnp.float32),
      compiler_params=pltpu.CompilerParams(use_tc_tiling_on_sc=False),
      scratch_types=[
          pltpu.VMEM_SHARED((N, width), jnp.float32),
          pltpu.VMEM((IB, KCH), jnp.int32),
          pltpu.VMEM((IB, KCH), jnp.int32),
          pltpu.VMEM((KCH, width), jnp.float32),
          pltpu.VMEM((KCH, 16), jnp.float32),
          pltpu.VMEM((KCH, width), jnp.float32),
          pltpu.SemaphoreType.DMA,
          pltpu.SemaphoreType.DMA,
      ],
  )
  def k(g_hbm, d_hbm, src_hbm, dst_hbm, z_hbm, out_hbm,
        acc, sidx, didx, grow, drow, msg, sem1, sem2):
    cid = lax.axis_index("c")
    sid = lax.axis_index("s")
    wid = cid * 16 + sid

    pltpu.sync_copy(z_hbm, acc.at[pl.ds(sid * rows_per_tile, rows_per_tile)])
    plsc.subcore_barrier()

    base = wid * NCHUNK
    nvec = width // 16

    def block_body(jb, carry):
      pltpu.sync_copy(src_hbm.at[pl.ds(base + jb * IB, IB)], sidx)
      pltpu.sync_copy(dst_hbm.at[pl.ds(base + jb * IB, IB)], didx)
      lax.fori_loop(0, IB, chunk_body, 0)
      return carry

    def chunk_body(j, carry):
      cp1 = pltpu.async_copy(g_hbm.at[sidx.at[j]], grow, sem1)
      cp2 = pltpu.async_copy(d_hbm.at[didx.at[j]], drow, sem2)
      cp1.wait()
      cp2.wait()

      def edge_body(e, c2):
        z = grow[e, pl.ds(0, 16)] + drow[e, pl.ds(0, 16)]
        z = jnp.where(z >= 0.0, z, z * jnp.float32(0.2))
        w = jnp.exp(z)
        msg[e, pl.ds(0, 16)] = w
        for kk in range(1, nvec):
          lane = (kk - 1) if heads == HEADS else 0
          ws = w[lane]
          msg[e, pl.ds(16 * kk, 16)] = grow[e, pl.ds(16 * kk, 16)] * ws
        return c2

      lax.fori_loop(0, KCH, edge_body, 0, unroll=2)
      pltpu.sync_copy(msg, acc.at[didx.at[j]], add=True)
      return carry

    lax.fori_loop(0, NCHUNK // IB, block_body, 0)
    plsc.subcore_barrier()
    pltpu.sync_copy(
        acc.at[pl.ds(sid * rows_per_tile, rows_per_tile)],
        out_hbm.at[cid, pl.ds(sid * rows_per_tile, rows_per_tile)])

  return k


def _tc1_kernel(x_blk, c_blk, w1x, w1c, p1, pd1, g_out, d_out):
  h1 = jnp.dot(x_blk[...], w1x[...], precision=lax.Precision.HIGHEST)
  h1 = h1 + jnp.dot(c_blk[...], w1c[...], precision=lax.Precision.HIGHEST)
  g_out[...] = jnp.dot(h1, p1[...], precision=lax.Precision.HIGHEST)
  d_out[...] = jnp.dot(h1, pd1[...], precision=lax.Precision.HIGHEST)


def _tc2_kernel(p_blk, sm, rm, b1_blk, w2, p2, pd2, g_out, d_out):
  accs = p_blk[0] + p_blk[1]
  den = jnp.dot(accs, rm[...], precision=lax.Precision.HIGHEST)
  msgs = jnp.dot(accs, sm[...], precision=lax.Precision.HIGHEST)
  h_mid = jnp.maximum(msgs / (den + 1e-16) + b1_blk[...], 0.0)
  h2 = jnp.dot(h_mid, w2[...], precision=lax.Precision.HIGHEST)
  g_out[...] = jnp.dot(h2, p2[...], precision=lax.Precision.HIGHEST)
  d_out[...] = jnp.dot(h2, pd2[...], precision=lax.Precision.HIGHEST)


def _tc3_kernel(p_blk, sm, rm, b2_blk, out_ref):
  accs = p_blk[0] + p_blk[1]
  den = jnp.dot(accs, rm[...], precision=lax.Precision.HIGHEST)
  msgs = jnp.dot(accs, sm[...], precision=lax.Precision.HIGHEST)
  out_ref[...] = msgs / (den + 1e-16) + b2_blk[...]


def kernel(x, edge_index, comm_ids, comm_table, W1, a_src1, a_dst1, b1,
           W2, a_src2, a_dst2, b2):
  f32 = jnp.float32

  w1x = W1[:D, :]
  w1c = jnp.zeros((16, W1COLS), f32).at[:CD, :].set(W1[D:, :])

  eye_h = jnp.eye(W1COLS, dtype=f32)
  a_s1_cols = jnp.zeros((W1COLS, 16), f32)
  a_d1_cols = jnp.zeros((W1COLS, 16), f32)
  for h in range(HEADS):
    a_s1_cols = a_s1_cols.at[h * HID:(h + 1) * HID, h].set(a_src1[h])
    a_d1_cols = a_d1_cols.at[h * HID:(h + 1) * HID, h].set(a_dst1[h])
  p1 = jnp.concatenate([a_s1_cols, eye_h], axis=1)
  pd1 = a_d1_cols

  sm1 = jnp.zeros((GW1, W1COLS), f32).at[16:, :].set(eye_h)
  rm1_np = np.zeros((GW1, W1COLS), np.float32)
  for h in range(HEADS):
    rm1_np[h, h * HID:(h + 1) * HID] = 1.0
  rm1 = jnp.asarray(rm1_np)

  eye_o = jnp.eye(OUT, dtype=f32)
  p2 = jnp.concatenate(
      [jnp.zeros((OUT, 16), f32).at[:, 0].set(a_src2[0]), eye_o], axis=1)
  pd2 = jnp.zeros((OUT, 16), f32).at[:, 0].set(a_dst2[0])
  sm2 = jnp.zeros((GW2, OUT), f32).at[16:, :].set(eye_o)
  rm2 = jnp.zeros((GW2, OUT), f32).at[0, :].set(1.0)

  src2d = edge_index[0].reshape(E // KCH, KCH)
  dst2d = edge_index[1].reshape(E // KCH, KCH)

  n_pad = 10240
  ids_pad = jnp.zeros((n_pad,), jnp.int32).at[:N].set(comm_ids)
  ids2d = ids_pad.reshape(n_pad // KCH, KCH)
  ctab_pad = jnp.zeros((NC, 16), f32).at[:, :CD].set(comm_table)

  z1 = jnp.zeros((N // 16, GW1), f32)
  z2 = jnp.zeros((N // 16, GW2), f32)

  c_full = _comm_gather_kernel(n_pad)(ids2d, ctab_pad)
  c_nodes = c_full[:N]

  nb = 5
  rb = N // nb
  g1, d1 = pl.pallas_call(
      _tc1_kernel,
      grid=(nb,),
      in_specs=[
          pl.BlockSpec((rb, D), lambda i: (i, 0)),
          pl.BlockSpec((rb, 16), lambda i: (i, 0)),
          pl.BlockSpec((D, W1COLS), lambda i: (0, 0)),
          pl.BlockSpec((16, W1COLS), lambda i: (0, 0)),
          pl.BlockSpec((W1COLS, GW1), lambda i: (0, 0)),
          pl.BlockSpec((W1COLS, 16), lambda i: (0, 0)),
      ],
      out_specs=[
          pl.BlockSpec((rb, GW1), lambda i: (i, 0)),
          pl.BlockSpec((rb, 16), lambda i: (i, 0)),
      ],
      out_shape=[
          jax.ShapeDtypeStruct((N, GW1), f32),
          jax.ShapeDtypeStruct((N, 16), f32),
      ],
  )(x, c_nodes, w1x, w1c, p1, pd1)

  part1 = _edge_kernel(GW1, HEADS)(g1, d1, src2d, dst2d, z1)

  g2, d2 = pl.pallas_call(
      _tc2_kernel,
      grid=(nb,),
      in_specs=[
          pl.BlockSpec((2, rb, GW1), lambda i: (0, i, 0)),
          pl.BlockSpec((GW1, W1COLS), lambda i: (0, 0)),
          pl.BlockSpec((GW1, W1COLS), lambda i: (0, 0)),
          pl.BlockSpec((1, W1COLS), lambda i: (0, 0)),
          pl.BlockSpec((W1COLS, OUT), lambda i: (0, 0)),
          pl.BlockSpec((OUT, GW2), lambda i: (0, 0)),
          pl.BlockSpec((OUT, 16), lambda i: (0, 0)),
      ],
      out_specs=[
          pl.BlockSpec((rb, GW2), lambda i: (i, 0)),
          pl.BlockSpec((rb, 16), lambda i: (i, 0)),
      ],
      out_shape=[
          jax.ShapeDtypeStruct((N, GW2), f32),
          jax.ShapeDtypeStruct((N, 16), f32),
      ],
  )(part1, sm1, rm1, b1.reshape(1, W1COLS), W2, p2, pd2)

  part2 = _edge_kernel(GW2, 1)(g2, d2, src2d, dst2d, z2)

  out = pl.pallas_call(
      _tc3_kernel,
      grid=(nb,),
      in_specs=[
          pl.BlockSpec((2, rb, GW2), lambda i: (0, i, 0)),
          pl.BlockSpec((GW2, OUT), lambda i: (0, 0)),
          pl.BlockSpec((GW2, OUT), lambda i: (0, 0)),
          pl.BlockSpec((1, OUT), lambda i: (0, 0)),
      ],
      out_specs=pl.BlockSpec((rb, OUT), lambda i: (i, 0)),
      out_shape=jax.ShapeDtypeStruct((N, OUT), f32),
  )(part2, sm2, rm2, b2.reshape(1, OUT))

  return out


# speedup vs baseline: 1.4831x; 1.4831x over previous
import functools

import jax
import jax.numpy as jnp
import numpy as np
from jax import lax
from jax.experimental import pallas as pl
from jax.experimental.pallas import tpu as pltpu
from jax.experimental.pallas import tpu_sc as plsc

N = 10000
E = 320000
D = 128
NC = 1000
CD = 8
HEADS = 8
HID = 16
OUT = 64

NWORKERS = 32
KCH = 80
EPW = E // NWORKERS
NCHUNK = EPW // KCH
IB = 25
W1COLS = HEADS * HID
GW1 = 16 + W1COLS
GW2 = 16 + OUT


def _comm_gather_kernel(n_pad):
  per_w = n_pad // NWORKERS
  steps = per_w // KCH
  mesh = plsc.VectorSubcoreMesh(core_axis_name="c", subcore_axis_name="s")

  @functools.partial(
      pl.kernel,
      mesh=mesh,
      out_type=jax.ShapeDtypeStruct((n_pad, 16), jnp.float32),
      compiler_params=pltpu.CompilerParams(use_tc_tiling_on_sc=False),
      scratch_types=[
          pltpu.VMEM((steps, KCH), jnp.int32),
          pltpu.VMEM((KCH, 16), jnp.float32),
          pltpu.SemaphoreType.DMA,
      ],
  )
  def k(ids_hbm, ctab_hbm, out_hbm, idxb, rows, sem):
    cid = lax.axis_index("c")
    sid = lax.axis_index("s")
    wid = cid * 16 + sid
    pltpu.sync_copy(ids_hbm.at[pl.ds(wid * steps, steps)], idxb)
    for kk in range(steps):
      pltpu.async_copy(ctab_hbm.at[idxb.at[kk]], rows, sem).wait()
      pltpu.sync_copy(rows, out_hbm.at[pl.ds(wid * per_w + kk * KCH, KCH)])

  return k


def _lane_bcast(v, lane):
  idx = jnp.full((16, 1), lane, jnp.int32)
  return lax.gather(
      v, idx,
      lax.GatherDimensionNumbers(
          offset_dims=(), collapsed_slice_dims=(0,), start_index_map=(0,)),
      (1,),
      mode=lax.GatherScatterMode.PROMISE_IN_BOUNDS)


def _edge_kernel(width, heads):
  rows_per_tile = N // 16
  mesh = plsc.VectorSubcoreMesh(core_axis_name="c", subcore_axis_name="s")

  @functools.partial(
      pl.kernel,
      mesh=mesh,
      out_type=jax.ShapeDtypeStruct((2, N, width), jnp.float32),
      compiler_params=pltpu.CompilerParams(use_tc_tiling_on_sc=False),
      scratch_types=[
          pltpu.VMEM_SHARED((N, width), jnp.float32),
          pltpu.VMEM((IB, KCH), jnp.int32),
          pltpu.VMEM((IB, KCH), jnp.int32),
          pltpu.VMEM((KCH, width), jnp.float32),
          pltpu.VMEM((KCH, 16), jnp.float32),
          pltpu.VMEM((KCH, width), jnp.float32),
          pltpu.SemaphoreType.DMA,
          pltpu.SemaphoreType.DMA,
      ],
  )
  def k(g_hbm, d_hbm, src_hbm, dst_hbm, z_hbm, out_hbm,
        acc, sidx, didx, grow, drow, msg, sem1, sem2):
    cid = lax.axis_index("c")
    sid = lax.axis_index("s")
    wid = cid * 16 + sid

    pltpu.sync_copy(z_hbm, acc.at[pl.ds(sid * rows_per_tile, rows_per_tile)])
    plsc.subcore_barrier()

    base = wid * NCHUNK
    nvec = width // 16

    def block_body(jb, carry):
      pltpu.sync_copy(src_hbm.at[pl.ds(base + jb * IB, IB)], sidx)
      pltpu.sync_copy(dst_hbm.at[pl.ds(base + jb * IB, IB)], didx)
      lax.fori_loop(0, IB, chunk_body, 0)
      return carry

    def chunk_body(j, carry):
      cp1 = pltpu.async_copy(g_hbm.at[sidx.at[j]], grow, sem1)
      cp2 = pltpu.async_copy(d_hbm.at[didx.at[j]], drow, sem2)
      cp1.wait()
      cp2.wait()

      def edge_body(e, c2):
        z = grow[e, pl.ds(0, 16)] + drow[e, pl.ds(0, 16)]
        z = jnp.where(z >= 0.0, z, z * jnp.float32(0.2))
        w = jnp.exp(z)
        msg[e, pl.ds(0, 16)] = w
        for kk in range(1, nvec):
          lane = (kk - 1) if heads == HEADS else 0
          ws = w[lane]
          msg[e, pl.ds(16 * kk, 16)] = grow[e, pl.ds(16 * kk, 16)] * ws
        return c2

      lax.fori_loop(0, KCH, edge_body, 0)
      pltpu.sync_copy(msg, acc.at[didx.at[j]], add=True)
      return carry

    lax.fori_loop(0, NCHUNK // IB, block_body, 0)
    plsc.subcore_barrier()
    pltpu.sync_copy(
        acc.at[pl.ds(sid * rows_per_tile, rows_per_tile)],
        out_hbm.at[cid, pl.ds(sid * rows_per_tile, rows_per_tile)])

  return k


def _tc1_kernel(x_blk, c_blk, w1x, w1c, p1, pd1, g_out, d_out):
  h1 = jnp.dot(x_blk[...], w1x[...], precision=lax.Precision.HIGHEST)
  h1 = h1 + jnp.dot(c_blk[...], w1c[...], precision=lax.Precision.HIGHEST)
  g_out[...] = jnp.dot(h1, p1[...], precision=lax.Precision.HIGHEST)
  d_out[...] = jnp.dot(h1, pd1[...], precision=lax.Precision.HIGHEST)


def _tc2_kernel(p_blk, sm, rm, b1_blk, w2, p2, pd2, g_out, d_out):
  accs = p_blk[0] + p_blk[1]
  den = jnp.dot(accs, rm[...], precision=lax.Precision.HIGHEST)
  msgs = jnp.dot(accs, sm[...], precision=lax.Precision.HIGHEST)
  h_mid = jnp.maximum(msgs / (den + 1e-16) + b1_blk[...], 0.0)
  h2 = jnp.dot(h_mid, w2[...], precision=lax.Precision.HIGHEST)
  g_out[...] = jnp.dot(h2, p2[...], precision=lax.Precision.HIGHEST)
  d_out[...] = jnp.dot(h2, pd2[...], precision=lax.Precision.HIGHEST)


def _tc3_kernel(p_blk, sm, rm, b2_blk, out_ref):
  accs = p_blk[0] + p_blk[1]
  den = jnp.dot(accs, rm[...], precision=lax.Precision.HIGHEST)
  msgs = jnp.dot(accs, sm[...], precision=lax.Precision.HIGHEST)
  out_ref[...] = msgs / (den + 1e-16) + b2_blk[...]


def kernel(x, edge_index, comm_ids, comm_table, W1, a_src1, a_dst1, b1,
           W2, a_src2, a_dst2, b2):
  f32 = jnp.float32

  w1x = W1[:D, :]
  w1c = jnp.zeros((16, W1COLS), f32).at[:CD, :].set(W1[D:, :])

  eye_h = jnp.eye(W1COLS, dtype=f32)
  a_s1_cols = jnp.zeros((W1COLS, 16), f32)
  a_d1_cols = jnp.zeros((W1COLS, 16), f32)
  for h in range(HEADS):
    a_s1_cols = a_s1_cols.at[h * HID:(h + 1) * HID, h].set(a_src1[h])
    a_d1_cols = a_d1_cols.at[h * HID:(h + 1) * HID, h].set(a_dst1[h])
  p1 = jnp.concatenate([a_s1_cols, eye_h], axis=1)
  pd1 = a_d1_cols

  sm1 = jnp.zeros((GW1, W1COLS), f32).at[16:, :].set(eye_h)
  rm1_np = np.zeros((GW1, W1COLS), np.float32)
  for h in range(HEADS):
    rm1_np[h, h * HID:(h + 1) * HID] = 1.0
  rm1 = jnp.asarray(rm1_np)

  eye_o = jnp.eye(OUT, dtype=f32)
  p2 = jnp.concatenate(
      [jnp.zeros((OUT, 16), f32).at[:, 0].set(a_src2[0]), eye_o], axis=1)
  pd2 = jnp.zeros((OUT, 16), f32).at[:, 0].set(a_dst2[0])
  sm2 = jnp.zeros((GW2, OUT), f32).at[16:, :].set(eye_o)
  rm2 = jnp.zeros((GW2, OUT), f32).at[0, :].set(1.0)

  src2d = edge_index[0].reshape(E // KCH, KCH)
  dst2d = edge_index[1].reshape(E // KCH, KCH)

  n_pad = 10240
  ids_pad = jnp.zeros((n_pad,), jnp.int32).at[:N].set(comm_ids)
  ids2d = ids_pad.reshape(n_pad // KCH, KCH)
  ctab_pad = jnp.zeros((NC, 16), f32).at[:, :CD].set(comm_table)

  z1 = jnp.zeros((N // 16, GW1), f32)
  z2 = jnp.zeros((N // 16, GW2), f32)

  c_full = _comm_gather_kernel(n_pad)(ids2d, ctab_pad)
  c_nodes = c_full[:N]

  nb = 5
  rb = N // nb
  g1, d1 = pl.pallas_call(
      _tc1_kernel,
      grid=(nb,),
      in_specs=[
          pl.BlockSpec((rb, D), lambda i: (i, 0)),
          pl.BlockSpec((rb, 16), lambda i: (i, 0)),
          pl.BlockSpec((D, W1COLS), lambda i: (0, 0)),
          pl.BlockSpec((16, W1COLS), lambda i: (0, 0)),
          pl.BlockSpec((W1COLS, GW1), lambda i: (0, 0)),
          pl.BlockSpec((W1COLS, 16), lambda i: (0, 0)),
      ],
      out_specs=[
          pl.BlockSpec((rb, GW1), lambda i: (i, 0)),
          pl.BlockSpec((rb, 16), lambda i: (i, 0)),
      ],
      out_shape=[
          jax.ShapeDtypeStruct((N, GW1), f32),
          jax.ShapeDtypeStruct((N, 16), f32),
      ],
  )(x, c_nodes, w1x, w1c, p1, pd1)

  part1 = _edge_kernel(GW1, HEADS)(g1, d1, src2d, dst2d, z1)

  g2, d2 = pl.pallas_call(
      _tc2_kernel,
      grid=(nb,),
      in_specs=[
          pl.BlockSpec((2, rb, GW1), lambda i: (0, i, 0)),
          pl.BlockSpec((GW1, W1COLS), lambda i: (0, 0)),
          pl.BlockSpec((GW1, W1COLS), lambda i: (0, 0)),
          pl.BlockSpec((1, W1COLS), lambda i: (0, 0)),
          pl.BlockSpec((W1COLS, OUT), lambda i: (0, 0)),
          pl.BlockSpec((OUT, GW2), lambda i: (0, 0)),
          pl.BlockSpec((OUT, 16), lambda i: (0, 0)),
      ],
      out_specs=[
          pl.BlockSpec((rb, GW2), lambda i: (i, 0)),
          pl.BlockSpec((rb, 16), lambda i: (i, 0)),
      ],
      out_shape=[
          jax.ShapeDtypeStruct((N, GW2), f32),
          jax.ShapeDtypeStruct((N, 16), f32),
      ],
  )(part1, sm1, rm1, b1.reshape(1, W1COLS), W2, p2, pd2)

  part2 = _edge_kernel(GW2, 1)(g2, d2, src2d, dst2d, z2)

  out = pl.pallas_call(
      _tc3_kernel,
      grid=(nb,),
      in_specs=[
          pl.BlockSpec((2, rb, GW2), lambda i: (0, i, 0)),
          pl.BlockSpec((GW2, OUT), lambda i: (0, 0)),
          pl.BlockSpec((GW2, OUT), lambda i: (0, 0)),
          pl.BlockSpec((1, OUT), lambda i: (0, 0)),
      ],
      out_specs=pl.BlockSpec((rb, OUT), lambda i: (i, 0)),
      out_shape=jax.ShapeDtypeStruct((N, OUT), f32),
  )(part2, sm2, rm2, b2.reshape(1, OUT))

  return out


# speedup vs baseline: 2.0591x; 1.3884x over previous
import functools

import jax
import jax.numpy as jnp
import numpy as np
from jax import lax
from jax.experimental import pallas as pl
from jax.experimental.pallas import tpu as pltpu
from jax.experimental.pallas import tpu_sc as plsc

N = 10000
E = 320000
D = 128
NC = 1000
CD = 8
HEADS = 8
HID = 16
OUT = 64

NWORKERS = 32
KCH = 80
EPW = E // NWORKERS
NCHUNK = EPW // KCH
IB = 25
W1COLS = HEADS * HID
GW1 = 16 + W1COLS
GW2 = 16 + OUT


def _comm_gather_kernel(n_pad):
  per_w = n_pad // NWORKERS
  steps = per_w // KCH
  mesh = plsc.VectorSubcoreMesh(core_axis_name="c", subcore_axis_name="s")

  @functools.partial(
      pl.kernel,
      mesh=mesh,
      out_type=jax.ShapeDtypeStruct((n_pad, 16), jnp.float32),
      compiler_params=pltpu.CompilerParams(use_tc_tiling_on_sc=False),
      scratch_types=[
          pltpu.VMEM((steps, KCH), jnp.int32),
          pltpu.VMEM((KCH, 16), jnp.float32),
          pltpu.SemaphoreType.DMA,
      ],
  )
  def k(ids_hbm, ctab_hbm, out_hbm, idxb, rows, sem):
    cid = lax.axis_index("c")
    sid = lax.axis_index("s")
    wid = cid * 16 + sid
    pltpu.sync_copy(ids_hbm.at[pl.ds(wid * steps, steps)], idxb)
    for kk in range(steps):
      pltpu.async_copy(ctab_hbm.at[idxb.at[kk]], rows, sem).wait()
      pltpu.sync_copy(rows, out_hbm.at[pl.ds(wid * per_w + kk * KCH, KCH)])

  return k


def _lane_bcast(v, lane):
  idx = jnp.full((16, 1), lane, jnp.int32)
  return lax.gather(
      v, idx,
      lax.GatherDimensionNumbers(
          offset_dims=(), collapsed_slice_dims=(0,), start_index_map=(0,)),
      (1,),
      mode=lax.GatherScatterMode.PROMISE_IN_BOUNDS)


def _edge_kernel(width, heads):
  rows_per_tile = N // 16
  mesh = plsc.VectorSubcoreMesh(core_axis_name="c", subcore_axis_name="s")

  @functools.partial(
      pl.kernel,
      mesh=mesh,
      out_type=jax.ShapeDtypeStruct((2, N, width), jnp.float32),
      compiler_params=pltpu.CompilerParams(use_tc_tiling_on_sc=False),
      scratch_types=[
          pltpu.VMEM_SHARED((N, width), jnp.float32),
          pltpu.VMEM((IB, KCH), jnp.int32),
          pltpu.VMEM((IB, KCH), jnp.int32),
          pltpu.VMEM((KCH, width), jnp.float32),
          pltpu.VMEM((KCH, 16), jnp.float32),
          pltpu.VMEM((KCH, width), jnp.float32),
          pltpu.SemaphoreType.DMA,
          pltpu.SemaphoreType.DMA,
      ],
  )
  def k(g_hbm, d_hbm, src_hbm, dst_hbm, z_hbm, out_hbm,
        acc, sidx, didx, grow, drow, msg, sem1, sem2):
    cid = lax.axis_index("c")
    sid = lax.axis_index("s")
    wid = cid * 16 + sid

    pltpu.sync_copy(z_hbm, acc.at[pl.ds(sid * rows_per_tile, rows_per_tile)])
    plsc.subcore_barrier()

    base = wid * NCHUNK
    nvec = width // 16

    def block_body(jb, carry):
      pltpu.sync_copy(src_hbm.at[pl.ds(base + jb * IB, IB)], sidx)
      pltpu.sync_copy(dst_hbm.at[pl.ds(base + jb * IB, IB)], didx)
      lax.fori_loop(0, IB, chunk_body, 0)
      return carry

    def chunk_body(j, carry):
      cp1 = pltpu.async_copy(g_hbm.at[sidx.at[j]], grow, sem1)
      cp2 = pltpu.async_copy(d_hbm.at[didx.at[j]], drow, sem2)
      cp1.wait()
      cp2.wait()

      @plsc.parallel_loop(0, KCH)
      def edge_body(e):
        z = grow[e, pl.ds(0, 16)] + drow[e, pl.ds(0, 16)]
        z = jnp.where(z >= 0.0, z, z * jnp.float32(0.2))
        w = jnp.exp(z)
        msg[e, pl.ds(0, 16)] = w
        for kk in range(1, nvec):
          lane = (kk - 1) if heads == HEADS else 0
          ws = w[lane]
          msg[e, pl.ds(16 * kk, 16)] = grow[e, pl.ds(16 * kk, 16)] * ws
      pltpu.sync_copy(msg, acc.at[didx.at[j]], add=True)
      return carry

    lax.fori_loop(0, NCHUNK // IB, block_body, 0)
    plsc.subcore_barrier()
    pltpu.sync_copy(
        acc.at[pl.ds(sid * rows_per_tile, rows_per_tile)],
        out_hbm.at[cid, pl.ds(sid * rows_per_tile, rows_per_tile)])

  return k


def _tc1_kernel(x_blk, c_blk, w1x, w1c, p1, pd1, g_out, d_out):
  h1 = jnp.dot(x_blk[...], w1x[...], precision=lax.Precision.HIGHEST)
  h1 = h1 + jnp.dot(c_blk[...], w1c[...], precision=lax.Precision.HIGHEST)
  g_out[...] = jnp.dot(h1, p1[...], precision=lax.Precision.HIGHEST)
  d_out[...] = jnp.dot(h1, pd1[...], precision=lax.Precision.HIGHEST)


def _tc2_kernel(p_blk, sm, rm, b1_blk, w2, p2, pd2, g_out, d_out):
  accs = p_blk[0] + p_blk[1]
  den = jnp.dot(accs, rm[...], precision=lax.Precision.HIGHEST)
  msgs = jnp.dot(accs, sm[...], precision=lax.Precision.HIGHEST)
  h_mid = jnp.maximum(msgs / (den + 1e-16) + b1_blk[...], 0.0)
  h2 = jnp.dot(h_mid, w2[...], precision=lax.Precision.HIGHEST)
  g_out[...] = jnp.dot(h2, p2[...], precision=lax.Precision.HIGHEST)
  d_out[...] = jnp.dot(h2, pd2[...], precision=lax.Precision.HIGHEST)


def _tc3_kernel(p_blk, sm, rm, b2_blk, out_ref):
  accs = p_blk[0] + p_blk[1]
  den = jnp.dot(accs, rm[...], precision=lax.Precision.HIGHEST)
  msgs = jnp.dot(accs, sm[...], precision=lax.Precision.HIGHEST)
  out_ref[...] = msgs / (den + 1e-16) + b2_blk[...]


def kernel(x, edge_index, comm_ids, comm_table, W1, a_src1, a_dst1, b1,
           W2, a_src2, a_dst2, b2):
  f32 = jnp.float32

  w1x = W1[:D, :]
  w1c = jnp.zeros((16, W1COLS), f32).at[:CD, :].set(W1[D:, :])

  eye_h = jnp.eye(W1COLS, dtype=f32)
  a_s1_cols = jnp.zeros((W1COLS, 16), f32)
  a_d1_cols = jnp.zeros((W1COLS, 16), f32)
  for h in range(HEADS):
    a_s1_cols = a_s1_cols.at[h * HID:(h + 1) * HID, h].set(a_src1[h])
    a_d1_cols = a_d1_cols.at[h * HID:(h + 1) * HID, h].set(a_dst1[h])
  p1 = jnp.concatenate([a_s1_cols, eye_h], axis=1)
  pd1 = a_d1_cols

  sm1 = jnp.zeros((GW1, W1COLS), f32).at[16:, :].set(eye_h)
  rm1_np = np.zeros((GW1, W1COLS), np.float32)
  for h in range(HEADS):
    rm1_np[h, h * HID:(h + 1) * HID] = 1.0
  rm1 = jnp.asarray(rm1_np)

  eye_o = jnp.eye(OUT, dtype=f32)
  p2 = jnp.concatenate(
      [jnp.zeros((OUT, 16), f32).at[:, 0].set(a_src2[0]), eye_o], axis=1)
  pd2 = jnp.zeros((OUT, 16), f32).at[:, 0].set(a_dst2[0])
  sm2 = jnp.zeros((GW2, OUT), f32).at[16:, :].set(eye_o)
  rm2 = jnp.zeros((GW2, OUT), f32).at[0, :].set(1.0)

  src2d = edge_index[0].reshape(E // KCH, KCH)
  dst2d = edge_index[1].reshape(E // KCH, KCH)

  n_pad = 10240
  ids_pad = jnp.zeros((n_pad,), jnp.int32).at[:N].set(comm_ids)
  ids2d = ids_pad.reshape(n_pad // KCH, KCH)
  ctab_pad = jnp.zeros((NC, 16), f32).at[:, :CD].set(comm_table)

  z1 = jnp.zeros((N // 16, GW1), f32)
  z2 = jnp.zeros((N // 16, GW2), f32)

  c_full = _comm_gather_kernel(n_pad)(ids2d, ctab_pad)
  c_nodes = c_full[:N]

  nb = 5
  rb = N // nb
  g1, d1 = pl.pallas_call(
      _tc1_kernel,
      grid=(nb,),
      in_specs=[
          pl.BlockSpec((rb, D), lambda i: (i, 0)),
          pl.BlockSpec((rb, 16), lambda i: (i, 0)),
          pl.BlockSpec((D, W1COLS), lambda i: (0, 0)),
          pl.BlockSpec((16, W1COLS), lambda i: (0, 0)),
          pl.BlockSpec((W1COLS, GW1), lambda i: (0, 0)),
          pl.BlockSpec((W1COLS, 16), lambda i: (0, 0)),
      ],
      out_specs=[
          pl.BlockSpec((rb, GW1), lambda i: (i, 0)),
          pl.BlockSpec((rb, 16), lambda i: (i, 0)),
      ],
      out_shape=[
          jax.ShapeDtypeStruct((N, GW1), f32),
          jax.ShapeDtypeStruct((N, 16), f32),
      ],
  )(x, c_nodes, w1x, w1c, p1, pd1)

  part1 = _edge_kernel(GW1, HEADS)(g1, d1, src2d, dst2d, z1)

  g2, d2 = pl.pallas_call(
      _tc2_kernel,
      grid=(nb,),
      in_specs=[
          pl.BlockSpec((2, rb, GW1), lambda i: (0, i, 0)),
          pl.BlockSpec((GW1, W1COLS), lambda i: (0, 0)),
          pl.BlockSpec((GW1, W1COLS), lambda i: (0, 0)),
          pl.BlockSpec((1, W1COLS), lambda i: (0, 0)),
          pl.BlockSpec((W1COLS, OUT), lambda i: (0, 0)),
          pl.BlockSpec((OUT, GW2), lambda i: (0, 0)),
          pl.BlockSpec((OUT, 16), lambda i: (0, 0)),
      ],
      out_specs=[
          pl.BlockSpec((rb, GW2), lambda i: (i, 0)),
          pl.BlockSpec((rb, 16), lambda i: (i, 0)),
      ],
      out_shape=[
          jax.ShapeDtypeStruct((N, GW2), f32),
          jax.ShapeDtypeStruct((N, 16), f32),
      ],
  )(part1, sm1, rm1, b1.reshape(1, W1COLS), W2, p2, pd2)

  part2 = _edge_kernel(GW2, 1)(g2, d2, src2d, dst2d, z2)

  out = pl.pallas_call(
      _tc3_kernel,
      grid=(nb,),
      in_specs=[
          pl.BlockSpec((2, rb, GW2), lambda i: (0, i, 0)),
          pl.BlockSpec((GW2, OUT), lambda i: (0, 0)),
          pl.BlockSpec((GW2, OUT), lambda i: (0, 0)),
          pl.BlockSpec((1, OUT), lambda i: (0, 0)),
      ],
      out_specs=pl.BlockSpec((rb, OUT), lambda i: (i, 0)),
      out_shape=jax.ShapeDtypeStruct((N, OUT), f32),
  )(part2, sm2, rm2, b2.reshape(1, OUT))

  return out


# speedup vs baseline: 2.0606x; 1.0007x over previous
import functools

import jax
import jax.numpy as jnp
import numpy as np
from jax import lax
from jax.experimental import pallas as pl
from jax.experimental.pallas import tpu as pltpu
from jax.experimental.pallas import tpu_sc as plsc

N = 10000
E = 320000
D = 128
NC = 1000
CD = 8
HEADS = 8
HID = 16
OUT = 64

NWORKERS = 32
KCH = 80
EPW = E // NWORKERS
NCHUNK = EPW // KCH
IB = 25
W1COLS = HEADS * HID
GW1 = 16 + W1COLS
GW2 = 16 + OUT


def _comm_gather_kernel(n_pad):
  per_w = n_pad // NWORKERS
  steps = per_w // KCH
  mesh = plsc.VectorSubcoreMesh(core_axis_name="c", subcore_axis_name="s")

  @functools.partial(
      pl.kernel,
      mesh=mesh,
      out_type=jax.ShapeDtypeStruct((n_pad, 16), jnp.float32),
      compiler_params=pltpu.CompilerParams(use_tc_tiling_on_sc=False),
      scratch_types=[
          pltpu.VMEM((steps, KCH), jnp.int32),
          pltpu.VMEM((KCH, 16), jnp.float32),
          pltpu.SemaphoreType.DMA,
      ],
  )
  def k(ids_hbm, ctab_hbm, out_hbm, idxb, rows, sem):
    cid = lax.axis_index("c")
    sid = lax.axis_index("s")
    wid = cid * 16 + sid
    pltpu.sync_copy(ids_hbm.at[pl.ds(wid * steps, steps)], idxb)
    for kk in range(steps):
      pltpu.async_copy(ctab_hbm.at[idxb.at[kk]], rows, sem).wait()
      pltpu.sync_copy(rows, out_hbm.at[pl.ds(wid * per_w + kk * KCH, KCH)])

  return k


def _lane_bcast(v, lane):
  idx = jnp.full((16, 1), lane, jnp.int32)
  return lax.gather(
      v, idx,
      lax.GatherDimensionNumbers(
          offset_dims=(), collapsed_slice_dims=(0,), start_index_map=(0,)),
      (1,),
      mode=lax.GatherScatterMode.PROMISE_IN_BOUNDS)


def _edge_kernel(width, heads):
  rows_per_tile = N // 16
  mesh = plsc.VectorSubcoreMesh(core_axis_name="c", subcore_axis_name="s")

  @functools.partial(
      pl.kernel,
      mesh=mesh,
      out_type=jax.ShapeDtypeStruct((2, N, width), jnp.float32),
      compiler_params=pltpu.CompilerParams(use_tc_tiling_on_sc=False),
      scratch_types=[
          pltpu.VMEM_SHARED((N, width), jnp.float32),
          pltpu.VMEM((IB, KCH), jnp.int32),
          pltpu.VMEM((IB, KCH), jnp.int32),
          pltpu.VMEM((KCH, width), jnp.float32),
          pltpu.VMEM((KCH, 16), jnp.float32),
          pltpu.VMEM((KCH, width), jnp.float32),
          pltpu.SemaphoreType.DMA,
          pltpu.SemaphoreType.DMA,
      ],
  )
  def k(g_hbm, d_hbm, src_hbm, dst_hbm, z_hbm, out_hbm,
        acc, sidx, didx, grow, drow, msg, sem1, sem2):
    cid = lax.axis_index("c")
    sid = lax.axis_index("s")
    wid = cid * 16 + sid

    pltpu.sync_copy(z_hbm, acc.at[pl.ds(sid * rows_per_tile, rows_per_tile)])
    plsc.subcore_barrier()

    base = wid * NCHUNK
    nvec = width // 16

    def block_body(jb, carry):
      pltpu.sync_copy(src_hbm.at[pl.ds(base + jb * IB, IB)], sidx)
      pltpu.sync_copy(dst_hbm.at[pl.ds(base + jb * IB, IB)], didx)
      lax.fori_loop(0, IB, chunk_body, 0)
      return carry

    def chunk_body(j, carry):
      cp1 = pltpu.async_copy(g_hbm.at[sidx.at[j]], grow, sem1)
      cp2 = pltpu.async_copy(d_hbm.at[didx.at[j]], drow, sem2)
      cp1.wait()
      cp2.wait()

      @plsc.parallel_loop(0, KCH)
      def edge_body(e):
        z = grow[e, pl.ds(0, 16)] + drow[e, pl.ds(0, 16)]
        z = jnp.where(z >= 0.0, z, z * jnp.float32(0.2))
        w = jnp.exp(z)
        msg[e, pl.ds(0, 16)] = w
        for kk in range(1, nvec):
          lane = (kk - 1) if heads == HEADS else 0
          wb = _lane_bcast(w, lane)
          msg[e, pl.ds(16 * kk, 16)] = grow[e, pl.ds(16 * kk, 16)] * wb
      pltpu.sync_copy(msg, acc.at[didx.at[j]], add=True)
      return carry

    lax.fori_loop(0, NCHUNK // IB, block_body, 0)
    plsc.subcore_barrier()
    pltpu.sync_copy(
        acc.at[pl.ds(sid * rows_per_tile, rows_per_tile)],
        out_hbm.at[cid, pl.ds(sid * rows_per_tile, rows_per_tile)])

  return k


def _tc1_kernel(x_blk, c_blk, w1x, w1c, p1, pd1, g_out, d_out):
  h1 = jnp.dot(x_blk[...], w1x[...], precision=lax.Precision.HIGHEST)
  h1 = h1 + jnp.dot(c_blk[...], w1c[...], precision=lax.Precision.HIGHEST)
  g_out[...] = jnp.dot(h1, p1[...], precision=lax.Precision.HIGHEST)
  d_out[...] = jnp.dot(h1, pd1[...], precision=lax.Precision.HIGHEST)


def _tc2_kernel(p_blk, sm, rm, b1_blk, w2, p2, pd2, g_out, d_out):
  accs = p_blk[0] + p_blk[1]
  den = jnp.dot(accs, rm[...], precision=lax.Precision.HIGHEST)
  msgs = jnp.dot(accs, sm[...], precision=lax.Precision.HIGHEST)
  h_mid = jnp.maximum(msgs / (den + 1e-16) + b1_blk[...], 0.0)
  h2 = jnp.dot(h_mid, w2[...], precision=lax.Precision.HIGHEST)
  g_out[...] = jnp.dot(h2, p2[...], precision=lax.Precision.HIGHEST)
  d_out[...] = jnp.dot(h2, pd2[...], precision=lax.Precision.HIGHEST)


def _tc3_kernel(p_blk, sm, rm, b2_blk, out_ref):
  accs = p_blk[0] + p_blk[1]
  den = jnp.dot(accs, rm[...], precision=lax.Precision.HIGHEST)
  msgs = jnp.dot(accs, sm[...], precision=lax.Precision.HIGHEST)
  out_ref[...] = msgs / (den + 1e-16) + b2_blk[...]


def kernel(x, edge_index, comm_ids, comm_table, W1, a_src1, a_dst1, b1,
           W2, a_src2, a_dst2, b2):
  f32 = jnp.float32

  w1x = W1[:D, :]
  w1c = jnp.zeros((16, W1COLS), f32).at[:CD, :].set(W1[D:, :])

  eye_h = jnp.eye(W1COLS, dtype=f32)
  a_s1_cols = jnp.zeros((W1COLS, 16), f32)
  a_d1_cols = jnp.zeros((W1COLS, 16), f32)
  for h in range(HEADS):
    a_s1_cols = a_s1_cols.at[h * HID:(h + 1) * HID, h].set(a_src1[h])
    a_d1_cols = a_d1_cols.at[h * HID:(h + 1) * HID, h].set(a_dst1[h])
  p1 = jnp.concatenate([a_s1_cols, eye_h], axis=1)
  pd1 = a_d1_cols

  sm1 = jnp.zeros((GW1, W1COLS), f32).at[16:, :].set(eye_h)
  rm1_np = np.zeros((GW1, W1COLS), np.float32)
  for h in range(HEADS):
    rm1_np[h, h * HID:(h + 1) * HID] = 1.0
  rm1 = jnp.asarray(rm1_np)

  eye_o = jnp.eye(OUT, dtype=f32)
  p2 = jnp.concatenate(
      [jnp.zeros((OUT, 16), f32).at[:, 0].set(a_src2[0]), eye_o], axis=1)
  pd2 = jnp.zeros((OUT, 16), f32).at[:, 0].set(a_dst2[0])
  sm2 = jnp.zeros((GW2, OUT), f32).at[16:, :].set(eye_o)
  rm2 = jnp.zeros((GW2, OUT), f32).at[0, :].set(1.0)

  src2d = edge_index[0].reshape(E // KCH, KCH)
  dst2d = edge_index[1].reshape(E // KCH, KCH)

  n_pad = 10240
  ids_pad = jnp.zeros((n_pad,), jnp.int32).at[:N].set(comm_ids)
  ids2d = ids_pad.reshape(n_pad // KCH, KCH)
  ctab_pad = jnp.zeros((NC, 16), f32).at[:, :CD].set(comm_table)

  z1 = jnp.zeros((N // 16, GW1), f32)
  z2 = jnp.zeros((N // 16, GW2), f32)

  c_full = _comm_gather_kernel(n_pad)(ids2d, ctab_pad)
  c_nodes = c_full[:N]

  nb = 5
  rb = N // nb
  g1, d1 = pl.pallas_call(
      _tc1_kernel,
      grid=(nb,),
      in_specs=[
          pl.BlockSpec((rb, D), lambda i: (i, 0)),
          pl.BlockSpec((rb, 16), lambda i: (i, 0)),
          pl.BlockSpec((D, W1COLS), lambda i: (0, 0)),
          pl.BlockSpec((16, W1COLS), lambda i: (0, 0)),
          pl.BlockSpec((W1COLS, GW1), lambda i: (0, 0)),
          pl.BlockSpec((W1COLS, 16), lambda i: (0, 0)),
      ],
      out_specs=[
          pl.BlockSpec((rb, GW1), lambda i: (i, 0)),
          pl.BlockSpec((rb, 16), lambda i: (i, 0)),
      ],
      out_shape=[
          jax.ShapeDtypeStruct((N, GW1), f32),
          jax.ShapeDtypeStruct((N, 16), f32),
      ],
  )(x, c_nodes, w1x, w1c, p1, pd1)

  part1 = _edge_kernel(GW1, HEADS)(g1, d1, src2d, dst2d, z1)

  g2, d2 = pl.pallas_call(
      _tc2_kernel,
      grid=(nb,),
      in_specs=[
          pl.BlockSpec((2, rb, GW1), lambda i: (0, i, 0)),
          pl.BlockSpec((GW1, W1COLS), lambda i: (0, 0)),
          pl.BlockSpec((GW1, W1COLS), lambda i: (0, 0)),
          pl.BlockSpec((1, W1COLS), lambda i: (0, 0)),
          pl.BlockSpec((W1COLS, OUT), lambda i: (0, 0)),
          pl.BlockSpec((OUT, GW2), lambda i: (0, 0)),
          pl.BlockSpec((OUT, 16), lambda i: (0, 0)),
      ],
      out_specs=[
          pl.BlockSpec((rb, GW2), lambda i: (i, 0)),
          pl.BlockSpec((rb, 16), lambda i: (i, 0)),
      ],
      out_shape=[
          jax.ShapeDtypeStruct((N, GW2), f32),
          jax.ShapeDtypeStruct((N, 16), f32),
      ],
  )(part1, sm1, rm1, b1.reshape(1, W1COLS), W2, p2, pd2)

  part2 = _edge_kernel(GW2, 1)(g2, d2, src2d, dst2d, z2)

  out = pl.pallas_call(
      _tc3_kernel,
      grid=(nb,),
      in_specs=[
          pl.BlockSpec((2, rb, GW2), lambda i: (0, i, 0)),
          pl.BlockSpec((GW2, OUT), lambda i: (0, 0)),
          pl.BlockSpec((GW2, OUT), lambda i: (0, 0)),
          pl.BlockSpec((1, OUT), lambda i: (0, 0)),
      ],
      out_specs=pl.BlockSpec((rb, OUT), lambda i: (i, 0)),
      out_shape=jax.ShapeDtypeStruct((N, OUT), f32),
  )(part2, sm2, rm2, b2.reshape(1, OUT))

  return out


# speedup vs baseline: 2.3178x; 1.1248x over previous
import functools

import jax
import jax.numpy as jnp
import numpy as np
from jax import lax
from jax.experimental import pallas as pl
from jax.experimental.pallas import tpu as pltpu
from jax.experimental.pallas import tpu_sc as plsc

N = 10000
E = 320000
D = 128
NC = 1000
CD = 8
HEADS = 8
HID = 16
OUT = 64

NWORKERS = 32
KCH = 80
EPW = E // NWORKERS
NCHUNK = EPW // KCH
IB = 25
W1COLS = HEADS * HID
GW1 = 16 + W1COLS
GW2 = 16 + OUT


def _comm_gather_kernel(n_pad):
  per_w = n_pad // NWORKERS
  steps = per_w // KCH
  mesh = plsc.VectorSubcoreMesh(core_axis_name="c", subcore_axis_name="s")

  @functools.partial(
      pl.kernel,
      mesh=mesh,
      out_type=jax.ShapeDtypeStruct((n_pad, 16), jnp.float32),
      compiler_params=pltpu.CompilerParams(use_tc_tiling_on_sc=False),
      scratch_types=[
          pltpu.VMEM((steps, KCH), jnp.int32),
          pltpu.VMEM((KCH, 16), jnp.float32),
          pltpu.SemaphoreType.DMA,
      ],
  )
  def k(ids_hbm, ctab_hbm, out_hbm, idxb, rows, sem):
    cid = lax.axis_index("c")
    sid = lax.axis_index("s")
    wid = cid * 16 + sid
    pltpu.sync_copy(ids_hbm.at[pl.ds(wid * steps, steps)], idxb)
    for kk in range(steps):
      pltpu.async_copy(ctab_hbm.at[idxb.at[kk]], rows, sem).wait()
      pltpu.sync_copy(rows, out_hbm.at[pl.ds(wid * per_w + kk * KCH, KCH)])

  return k


def _lane_bcast(v, lane):
  idx = jnp.full((16, 1), lane, jnp.int32)
  return lax.gather(
      v, idx,
      lax.GatherDimensionNumbers(
          offset_dims=(), collapsed_slice_dims=(0,), start_index_map=(0,)),
      (1,),
      mode=lax.GatherScatterMode.PROMISE_IN_BOUNDS)


def _edge_kernel(width, heads):
  rows_per_tile = N // 16
  mesh = plsc.VectorSubcoreMesh(core_axis_name="c", subcore_axis_name="s")

  @functools.partial(
      pl.kernel,
      mesh=mesh,
      out_type=jax.ShapeDtypeStruct((2, N, width), jnp.float32),
      compiler_params=pltpu.CompilerParams(use_tc_tiling_on_sc=False),
      scratch_types=[
          pltpu.VMEM_SHARED((N, width), jnp.float32),
          pltpu.VMEM((IB, KCH), jnp.int32),
          pltpu.VMEM((IB, KCH), jnp.int32),
          pltpu.VMEM((KCH, width), jnp.float32),
          pltpu.VMEM((KCH, width), jnp.float32),
          pltpu.VMEM((KCH, 16), jnp.float32),
          pltpu.VMEM((KCH, width), jnp.float32),
          pltpu.SemaphoreType.DMA,
          pltpu.SemaphoreType.DMA,
          pltpu.SemaphoreType.DMA,
      ],
  )
  def k(g_hbm, d_hbm, src_hbm, dst_hbm, z_hbm, out_hbm,
        acc, sidx, didx, grow0, grow1, drow, msg, sg0, sg1, sd):
    cid = lax.axis_index("c")
    sid = lax.axis_index("s")
    wid = cid * 16 + sid

    pltpu.sync_copy(z_hbm, acc.at[pl.ds(sid * rows_per_tile, rows_per_tile)])
    plsc.subcore_barrier()

    base = wid * NCHUNK
    nvec = width // 16

    def compute_chunk(gbuf, j):
      @plsc.parallel_loop(0, KCH)
      def edge_body(e):
        z = gbuf[e, pl.ds(0, 16)] + drow[e, pl.ds(0, 16)]
        z = jnp.where(z >= 0.0, z, z * jnp.float32(0.2))
        w = jnp.exp(z)
        msg[e, pl.ds(0, 16)] = w
        for kk in range(1, nvec):
          lane = (kk - 1) if heads == HEADS else 0
          wb = _lane_bcast(w, lane)
          msg[e, pl.ds(16 * kk, 16)] = gbuf[e, pl.ds(16 * kk, 16)] * wb

      pltpu.sync_copy(msg, acc.at[didx.at[j]], add=True)

    def block_body(jb, carry):
      pltpu.sync_copy(src_hbm.at[pl.ds(base + jb * IB, IB)], sidx)
      pltpu.sync_copy(dst_hbm.at[pl.ds(base + jb * IB, IB)], didx)
      pltpu.async_copy(g_hbm.at[sidx.at[0]], grow0, sg0)
      pltpu.async_copy(d_hbm.at[didx.at[0]], drow, sd)

      def pair_body(jj, c):
        j0 = 2 * jj
        pltpu.make_async_copy(g_hbm.at[sidx.at[j0]], grow0, sg0).wait()
        pltpu.async_copy(g_hbm.at[sidx.at[j0 + 1]], grow1, sg1)
        pltpu.make_async_copy(d_hbm.at[didx.at[j0]], drow, sd).wait()
        compute_chunk(grow0, j0)
        pltpu.async_copy(d_hbm.at[didx.at[j0 + 1]], drow, sd)
        pltpu.make_async_copy(g_hbm.at[sidx.at[j0 + 1]], grow1, sg1).wait()
        pltpu.async_copy(g_hbm.at[sidx.at[j0 + 2]], grow0, sg0)
        pltpu.make_async_copy(d_hbm.at[didx.at[j0 + 1]], drow, sd).wait()
        compute_chunk(grow1, j0 + 1)
        pltpu.async_copy(d_hbm.at[didx.at[j0 + 2]], drow, sd)
        return c

      lax.fori_loop(0, (IB - 1) // 2, pair_body, 0)
      pltpu.make_async_copy(g_hbm.at[sidx.at[IB - 1]], grow0, sg0).wait()
      pltpu.make_async_copy(d_hbm.at[didx.at[IB - 1]], drow, sd).wait()
      compute_chunk(grow0, IB - 1)
      return carry

    lax.fori_loop(0, NCHUNK // IB, block_body, 0)
    plsc.subcore_barrier()
    pltpu.sync_copy(
        acc.at[pl.ds(sid * rows_per_tile, rows_per_tile)],
        out_hbm.at[cid, pl.ds(sid * rows_per_tile, rows_per_tile)])

  return k


def _tc1_kernel(x_blk, c_blk, w1x, w1c, p1, pd1, g_out, d_out):
  h1 = jnp.dot(x_blk[...], w1x[...], precision=lax.Precision.HIGHEST)
  h1 = h1 + jnp.dot(c_blk[...], w1c[...], precision=lax.Precision.HIGHEST)
  g_out[...] = jnp.dot(h1, p1[...], precision=lax.Precision.HIGHEST)
  d_out[...] = jnp.dot(h1, pd1[...], precision=lax.Precision.HIGHEST)


def _tc2_kernel(p_blk, sm, rm, b1_blk, w2, p2, pd2, g_out, d_out):
  accs = p_blk[0] + p_blk[1]
  den = jnp.dot(accs, rm[...], precision=lax.Precision.HIGHEST)
  msgs = jnp.dot(accs, sm[...], precision=lax.Precision.HIGHEST)
  h_mid = jnp.maximum(msgs / (den + 1e-16) + b1_blk[...], 0.0)
  h2 = jnp.dot(h_mid, w2[...], precision=lax.Precision.HIGHEST)
  g_out[...] = jnp.dot(h2, p2[...], precision=lax.Precision.HIGHEST)
  d_out[...] = jnp.dot(h2, pd2[...], precision=lax.Precision.HIGHEST)


def _tc3_kernel(p_blk, sm, rm, b2_blk, out_ref):
  accs = p_blk[0] + p_blk[1]
  den = jnp.dot(accs, rm[...], precision=lax.Precision.HIGHEST)
  msgs = jnp.dot(accs, sm[...], precision=lax.Precision.HIGHEST)
  out_ref[...] = msgs / (den + 1e-16) + b2_blk[...]


def kernel(x, edge_index, comm_ids, comm_table, W1, a_src1, a_dst1, b1,
           W2, a_src2, a_dst2, b2):
  f32 = jnp.float32

  w1x = W1[:D, :]
  w1c = jnp.zeros((16, W1COLS), f32).at[:CD, :].set(W1[D:, :])

  eye_h = jnp.eye(W1COLS, dtype=f32)
  a_s1_cols = jnp.zeros((W1COLS, 16), f32)
  a_d1_cols = jnp.zeros((W1COLS, 16), f32)
  for h in range(HEADS):
    a_s1_cols = a_s1_cols.at[h * HID:(h + 1) * HID, h].set(a_src1[h])
    a_d1_cols = a_d1_cols.at[h * HID:(h + 1) * HID, h].set(a_dst1[h])
  p1 = jnp.concatenate([a_s1_cols, eye_h], axis=1)
  pd1 = a_d1_cols

  sm1 = jnp.zeros((GW1, W1COLS), f32).at[16:, :].set(eye_h)
  rm1_np = np.zeros((GW1, W1COLS), np.float32)
  for h in range(HEADS):
    rm1_np[h, h * HID:(h + 1) * HID] = 1.0
  rm1 = jnp.asarray(rm1_np)

  eye_o = jnp.eye(OUT, dtype=f32)
  p2 = jnp.concatenate(
      [jnp.zeros((OUT, 16), f32).at[:, 0].set(a_src2[0]), eye_o], axis=1)
  pd2 = jnp.zeros((OUT, 16), f32).at[:, 0].set(a_dst2[0])
  sm2 = jnp.zeros((GW2, OUT), f32).at[16:, :].set(eye_o)
  rm2 = jnp.zeros((GW2, OUT), f32).at[0, :].set(1.0)

  src2d = edge_index[0].reshape(E // KCH, KCH)
  dst2d = edge_index[1].reshape(E // KCH, KCH)

  n_pad = 10240
  ids_pad = jnp.zeros((n_pad,), jnp.int32).at[:N].set(comm_ids)
  ids2d = ids_pad.reshape(n_pad // KCH, KCH)
  ctab_pad = jnp.zeros((NC, 16), f32).at[:, :CD].set(comm_table)

  z1 = jnp.zeros((N // 16, GW1), f32)
  z2 = jnp.zeros((N // 16, GW2), f32)

  c_full = _comm_gather_kernel(n_pad)(ids2d, ctab_pad)
  c_nodes = c_full[:N]

  nb = 5
  rb = N // nb
  g1, d1 = pl.pallas_call(
      _tc1_kernel,
      grid=(nb,),
      in_specs=[
          pl.BlockSpec((rb, D), lambda i: (i, 0)),
          pl.BlockSpec((rb, 16), lambda i: (i, 0)),
          pl.BlockSpec((D, W1COLS), lambda i: (0, 0)),
          pl.BlockSpec((16, W1COLS), lambda i: (0, 0)),
          pl.BlockSpec((W1COLS, GW1), lambda i: (0, 0)),
          pl.BlockSpec((W1COLS, 16), lambda i: (0, 0)),
      ],
      out_specs=[
          pl.BlockSpec((rb, GW1), lambda i: (i, 0)),
          pl.BlockSpec((rb, 16), lambda i: (i, 0)),
      ],
      out_shape=[
          jax.ShapeDtypeStruct((N, GW1), f32),
          jax.ShapeDtypeStruct((N, 16), f32),
      ],
  )(x, c_nodes, w1x, w1c, p1, pd1)

  part1 = _edge_kernel(GW1, HEADS)(g1, d1, src2d, dst2d, z1)

  g2, d2 = pl.pallas_call(
      _tc2_kernel,
      grid=(nb,),
      in_specs=[
          pl.BlockSpec((2, rb, GW1), lambda i: (0, i, 0)),
          pl.BlockSpec((GW1, W1COLS), lambda i: (0, 0)),
          pl.BlockSpec((GW1, W1COLS), lambda i: (0, 0)),
          pl.BlockSpec((1, W1COLS), lambda i: (0, 0)),
          pl.BlockSpec((W1COLS, OUT), lambda i: (0, 0)),
          pl.BlockSpec((OUT, GW2), lambda i: (0, 0)),
          pl.BlockSpec((OUT, 16), lambda i: (0, 0)),
      ],
      out_specs=[
          pl.BlockSpec((rb, GW2), lambda i: (i, 0)),
          pl.BlockSpec((rb, 16), lambda i: (i, 0)),
      ],
      out_shape=[
          jax.ShapeDtypeStruct((N, GW2), f32),
          jax.ShapeDtypeStruct((N, 16), f32),
      ],
  )(part1, sm1, rm1, b1.reshape(1, W1COLS), W2, p2, pd2)

  part2 = _edge_kernel(GW2, 1)(g2, d2, src2d, dst2d, z2)

  out = pl.pallas_call(
      _tc3_kernel,
      grid=(nb,),
      in_specs=[
          pl.BlockSpec((2, rb, GW2), lambda i: (0, i, 0)),
          pl.BlockSpec((GW2, OUT), lambda i: (0, 0)),
          pl.BlockSpec((GW2, OUT), lambda i: (0, 0)),
          pl.BlockSpec((1, OUT), lambda i: (0, 0)),
      ],
      out_specs=pl.BlockSpec((rb, OUT), lambda i: (i, 0)),
      out_shape=jax.ShapeDtypeStruct((N, OUT), f32),
  )(part2, sm2, rm2, b2.reshape(1, OUT))

  return out


# speedup vs baseline: 2.3852x; 1.0291x over previous
import functools

import jax
import jax.numpy as jnp
import numpy as np
from jax import lax
from jax.experimental import pallas as pl
from jax.experimental.pallas import tpu as pltpu
from jax.experimental.pallas import tpu_sc as plsc

N = 10000
E = 320000
D = 128
NC = 1000
CD = 8
HEADS = 8
HID = 16
OUT = 64

NWORKERS = 32
KCH = 80
K1 = 50
K2 = 80
EPW = E // NWORKERS
NCHUNK = EPW // KCH
IB = 25
W1COLS = HEADS * HID
GW1 = 16 + W1COLS
GW2 = 16 + OUT


def _comm_gather_kernel(n_pad):
  per_w = n_pad // NWORKERS
  steps = per_w // KCH
  mesh = plsc.VectorSubcoreMesh(core_axis_name="c", subcore_axis_name="s")

  @functools.partial(
      pl.kernel,
      mesh=mesh,
      out_type=jax.ShapeDtypeStruct((n_pad, 16), jnp.float32),
      compiler_params=pltpu.CompilerParams(use_tc_tiling_on_sc=False),
      scratch_types=[
          pltpu.VMEM((steps, KCH), jnp.int32),
          pltpu.VMEM((KCH, 16), jnp.float32),
          pltpu.SemaphoreType.DMA,
      ],
  )
  def k(ids_hbm, ctab_hbm, out_hbm, idxb, rows, sem):
    cid = lax.axis_index("c")
    sid = lax.axis_index("s")
    wid = cid * 16 + sid
    pltpu.sync_copy(ids_hbm.at[pl.ds(wid * steps, steps)], idxb)
    for kk in range(steps):
      pltpu.async_copy(ctab_hbm.at[idxb.at[kk]], rows, sem).wait()
      pltpu.sync_copy(rows, out_hbm.at[pl.ds(wid * per_w + kk * KCH, KCH)])

  return k


def _lane_bcast(v, lane):
  idx = jnp.full((16, 1), lane, jnp.int32)
  return lax.gather(
      v, idx,
      lax.GatherDimensionNumbers(
          offset_dims=(), collapsed_slice_dims=(0,), start_index_map=(0,)),
      (1,),
      mode=lax.GatherScatterMode.PROMISE_IN_BOUNDS)


def _edge_kernel(width, heads, kch):
  rows_per_tile = N // 16
  nchunk = EPW // kch
  nblock = nchunk // IB
  mesh = plsc.VectorSubcoreMesh(core_axis_name="c", subcore_axis_name="s")

  @functools.partial(
      pl.kernel,
      mesh=mesh,
      out_type=jax.ShapeDtypeStruct((2, N, width), jnp.float32),
      compiler_params=pltpu.CompilerParams(use_tc_tiling_on_sc=False),
      scratch_types=[
          pltpu.VMEM_SHARED((N, width), jnp.float32),
          pltpu.VMEM((IB, kch), jnp.int32),
          pltpu.VMEM((IB, kch), jnp.int32),
          pltpu.VMEM((kch, width), jnp.float32),
          pltpu.VMEM((kch, width), jnp.float32),
          pltpu.VMEM((kch, 16), jnp.float32),
          pltpu.VMEM((kch, width), jnp.float32),
          pltpu.VMEM((kch, width), jnp.float32),
          pltpu.SemaphoreType.DMA,
          pltpu.SemaphoreType.DMA,
          pltpu.SemaphoreType.DMA,
          pltpu.SemaphoreType.DMA,
          pltpu.SemaphoreType.DMA,
      ],
  )
  def k(g_hbm, d_hbm, src_hbm, dst_hbm, z_hbm, out_hbm,
        acc, sidx, didx, grow0, grow1, drow, msg0, msg1,
        sg0, sg1, sd, ssc0, ssc1):
    cid = lax.axis_index("c")
    sid = lax.axis_index("s")
    wid = cid * 16 + sid

    pltpu.sync_copy(z_hbm, acc.at[pl.ds(sid * rows_per_tile, rows_per_tile)])
    plsc.subcore_barrier()

    base = wid * nchunk
    nvec = width // 16

    def compute_chunk(gbuf, mbuf):
      @plsc.parallel_loop(0, kch)
      def edge_body(e):
        z = gbuf[e, pl.ds(0, 16)] + drow[e, pl.ds(0, 16)]
        z = jnp.where(z >= 0.0, z, z * jnp.float32(0.2))
        w = jnp.exp(z)
        mbuf[e, pl.ds(0, 16)] = w
        for kk in range(1, nvec):
          lane = (kk - 1) if heads == HEADS else 0
          wb = _lane_bcast(w, lane)
          mbuf[e, pl.ds(16 * kk, 16)] = gbuf[e, pl.ds(16 * kk, 16)] * wb

    def fire_g(j, gbuf, sem):
      pltpu.async_copy(g_hbm.at[sidx.at[j]], gbuf, sem)

    def wait_g(j, gbuf, sem):
      pltpu.make_async_copy(g_hbm.at[sidx.at[j]], gbuf, sem).wait()

    def fire_d(j):
      pltpu.async_copy(d_hbm.at[didx.at[j]], drow, sd)

    def wait_d(j):
      pltpu.make_async_copy(d_hbm.at[didx.at[j]], drow, sd).wait()

    def fire_sct(j, mbuf, sem):
      pltpu.async_copy(mbuf, acc.at[didx.at[j]], sem, add=True)

    def wait_sct(j, mbuf, sem):
      pltpu.make_async_copy(mbuf, acc.at[didx.at[j]], sem).wait()

    def block_body(jb, carry):
      pltpu.sync_copy(src_hbm.at[pl.ds(base + jb * IB, IB)], sidx)
      pltpu.sync_copy(dst_hbm.at[pl.ds(base + jb * IB, IB)], didx)
      fire_g(0, grow0, sg0)
      fire_d(0)
      wait_g(0, grow0, sg0)
      fire_g(1, grow1, sg1)
      wait_d(0)
      compute_chunk(grow0, msg0)
      fire_d(1)
      fire_sct(0, msg0, ssc0)
      wait_g(1, grow1, sg1)
      fire_g(2, grow0, sg0)
      wait_d(1)
      compute_chunk(grow1, msg1)
      fire_d(2)
      fire_sct(1, msg1, ssc1)

      def pair_body(jj, c):
        j0 = 2 * jj + 2
        wait_g(j0, grow0, sg0)
        fire_g(j0 + 1, grow1, sg1)
        wait_sct(j0 - 2, msg0, ssc0)
        wait_d(j0)
        compute_chunk(grow0, msg0)
        fire_d(j0 + 1)
        fire_sct(j0, msg0, ssc0)
        wait_g(j0 + 1, grow1, sg1)
        fire_g(j0 + 2, grow0, sg0)
        wait_sct(j0 - 1, msg1, ssc1)
        wait_d(j0 + 1)
        compute_chunk(grow1, msg1)
        fire_d(j0 + 2)
        fire_sct(j0 + 1, msg1, ssc1)
        return c

      lax.fori_loop(0, (IB - 3) // 2, pair_body, 0)
      wait_g(IB - 1, grow0, sg0)
      wait_sct(IB - 3, msg0, ssc0)
      wait_d(IB - 1)
      compute_chunk(grow0, msg0)
      fire_sct(IB - 1, msg0, ssc0)
      wait_sct(IB - 1, msg0, ssc0)
      wait_sct(IB - 2, msg1, ssc1)
      return carry

    lax.fori_loop(0, nblock, block_body, 0)
    plsc.subcore_barrier()
    pltpu.sync_copy(
        acc.at[pl.ds(sid * rows_per_tile, rows_per_tile)],
        out_hbm.at[cid, pl.ds(sid * rows_per_tile, rows_per_tile)])

  return k


def _tc1_kernel(x_blk, c_blk, w1x, w1c, p1, pd1, g_out, d_out):
  h1 = jnp.dot(x_blk[...], w1x[...], precision=lax.Precision.HIGHEST)
  h1 = h1 + jnp.dot(c_blk[...], w1c[...], precision=lax.Precision.HIGHEST)
  g_out[...] = jnp.dot(h1, p1[...], precision=lax.Precision.HIGHEST)
  d_out[...] = jnp.dot(h1, pd1[...], precision=lax.Precision.HIGHEST)


def _tc2_kernel(p_blk, sm, rm, b1_blk, w2, p2, pd2, g_out, d_out):
  accs = p_blk[0] + p_blk[1]
  den = jnp.dot(accs, rm[...], precision=lax.Precision.HIGHEST)
  msgs = jnp.dot(accs, sm[...], precision=lax.Precision.HIGHEST)
  h_mid = jnp.maximum(msgs / (den + 1e-16) + b1_blk[...], 0.0)
  h2 = jnp.dot(h_mid, w2[...], precision=lax.Precision.HIGHEST)
  g_out[...] = jnp.dot(h2, p2[...], precision=lax.Precision.HIGHEST)
  d_out[...] = jnp.dot(h2, pd2[...], precision=lax.Precision.HIGHEST)


def _tc3_kernel(p_blk, sm, rm, b2_blk, out_ref):
  accs = p_blk[0] + p_blk[1]
  den = jnp.dot(accs, rm[...], precision=lax.Precision.HIGHEST)
  msgs = jnp.dot(accs, sm[...], precision=lax.Precision.HIGHEST)
  out_ref[...] = msgs / (den + 1e-16) + b2_blk[...]


def kernel(x, edge_index, comm_ids, comm_table, W1, a_src1, a_dst1, b1,
           W2, a_src2, a_dst2, b2):
  f32 = jnp.float32

  w1x = W1[:D, :]
  w1c = jnp.zeros((16, W1COLS), f32).at[:CD, :].set(W1[D:, :])

  eye_h = jnp.eye(W1COLS, dtype=f32)
  a_s1_cols = jnp.zeros((W1COLS, 16), f32)
  a_d1_cols = jnp.zeros((W1COLS, 16), f32)
  for h in range(HEADS):
    a_s1_cols = a_s1_cols.at[h * HID:(h + 1) * HID, h].set(a_src1[h])
    a_d1_cols = a_d1_cols.at[h * HID:(h + 1) * HID, h].set(a_dst1[h])
  p1 = jnp.concatenate([a_s1_cols, eye_h], axis=1)
  pd1 = a_d1_cols

  sm1 = jnp.zeros((GW1, W1COLS), f32).at[16:, :].set(eye_h)
  rm1_np = np.zeros((GW1, W1COLS), np.float32)
  for h in range(HEADS):
    rm1_np[h, h * HID:(h + 1) * HID] = 1.0
  rm1 = jnp.asarray(rm1_np)

  eye_o = jnp.eye(OUT, dtype=f32)
  p2 = jnp.concatenate(
      [jnp.zeros((OUT, 16), f32).at[:, 0].set(a_src2[0]), eye_o], axis=1)
  pd2 = jnp.zeros((OUT, 16), f32).at[:, 0].set(a_dst2[0])
  sm2 = jnp.zeros((GW2, OUT), f32).at[16:, :].set(eye_o)
  rm2 = jnp.zeros((GW2, OUT), f32).at[0, :].set(1.0)

  src1_2d = edge_index[0].reshape(E // K1, K1)
  dst1_2d = edge_index[1].reshape(E // K1, K1)
  src2_2d = edge_index[0].reshape(E // K2, K2)
  dst2_2d = edge_index[1].reshape(E // K2, K2)

  n_pad = 10240
  ids_pad = jnp.zeros((n_pad,), jnp.int32).at[:N].set(comm_ids)
  ids2d = ids_pad.reshape(n_pad // KCH, KCH)
  ctab_pad = jnp.zeros((NC, 16), f32).at[:, :CD].set(comm_table)

  z1 = jnp.zeros((N // 16, GW1), f32)
  z2 = jnp.zeros((N // 16, GW2), f32)

  c_full = _comm_gather_kernel(n_pad)(ids2d, ctab_pad)
  c_nodes = c_full[:N]

  nb = 5
  rb = N // nb
  g1, d1 = pl.pallas_call(
      _tc1_kernel,
      grid=(nb,),
      in_specs=[
          pl.BlockSpec((rb, D), lambda i: (i, 0)),
          pl.BlockSpec((rb, 16), lambda i: (i, 0)),
          pl.BlockSpec((D, W1COLS), lambda i: (0, 0)),
          pl.BlockSpec((16, W1COLS), lambda i: (0, 0)),
          pl.BlockSpec((W1COLS, GW1), lambda i: (0, 0)),
          pl.BlockSpec((W1COLS, 16), lambda i: (0, 0)),
      ],
      out_specs=[
          pl.BlockSpec((rb, GW1), lambda i: (i, 0)),
          pl.BlockSpec((rb, 16), lambda i: (i, 0)),
      ],
      out_shape=[
          jax.ShapeDtypeStruct((N, GW1), f32),
          jax.ShapeDtypeStruct((N, 16), f32),
      ],
  )(x, c_nodes, w1x, w1c, p1, pd1)

  part1 = _edge_kernel(GW1, HEADS, K1)(g1, d1, src1_2d, dst1_2d, z1)

  g2, d2 = pl.pallas_call(
      _tc2_kernel,
      grid=(nb,),
      in_specs=[
          pl.BlockSpec((2, rb, GW1), lambda i: (0, i, 0)),
          pl.BlockSpec((GW1, W1COLS), lambda i: (0, 0)),
          pl.BlockSpec((GW1, W1COLS), lambda i: (0, 0)),
          pl.BlockSpec((1, W1COLS), lambda i: (0, 0)),
          pl.BlockSpec((W1COLS, OUT), lambda i: (0, 0)),
          pl.BlockSpec((OUT, GW2), lambda i: (0, 0)),
          pl.BlockSpec((OUT, 16), lambda i: (0, 0)),
      ],
      out_specs=[
          pl.BlockSpec((rb, GW2), lambda i: (i, 0)),
          pl.BlockSpec((rb, 16), lambda i: (i, 0)),
      ],
      out_shape=[
          jax.ShapeDtypeStruct((N, GW2), f32),
          jax.ShapeDtypeStruct((N, 16), f32),
      ],
  )(part1, sm1, rm1, b1.reshape(1, W1COLS), W2, p2, pd2)

  part2 = _edge_kernel(GW2, 1, K2)(g2, d2, src2_2d, dst2_2d, z2)

  out = pl.pallas_call(
      _tc3_kernel,
      grid=(nb,),
      in_specs=[
          pl.BlockSpec((2, rb, GW2), lambda i: (0, i, 0)),
          pl.BlockSpec((GW2, OUT), lambda i: (0, 0)),
          pl.BlockSpec((GW2, OUT), lambda i: (0, 0)),
          pl.BlockSpec((1, OUT), lambda i: (0, 0)),
      ],
      out_specs=pl.BlockSpec((rb, OUT), lambda i: (i, 0)),
      out_shape=jax.ShapeDtypeStruct((N, OUT), f32),
  )(part2, sm2, rm2, b2.reshape(1, OUT))

  return out


# speedup vs baseline: 2.4045x; 1.0081x over previous
import functools

import jax
import jax.numpy as jnp
import numpy as np
from jax import lax
from jax.experimental import pallas as pl
from jax.experimental.pallas import tpu as pltpu
from jax.experimental.pallas import tpu_sc as plsc

N = 10000
E = 320000
D = 128
NC = 1000
CD = 8
HEADS = 8
HID = 16
OUT = 64

NWORKERS = 32
KCH = 80
K1 = 50
K2 = 80
EPW = E // NWORKERS
NCHUNK = EPW // KCH
IB = 25
W1COLS = HEADS * HID
GW1 = 16 + W1COLS
GW2 = 16 + OUT

_MASK1 = np.repeat(np.eye(HEADS, 16, dtype=np.float32), HID, axis=0)
_MASK2 = np.zeros((OUT, 16), np.float32)
_MASK2[:, 0] = 1.0
_EYE128 = np.eye(W1COLS, dtype=np.float32)
_EYE64 = np.eye(OUT, dtype=np.float32)
_Z8x128 = np.zeros((16 - CD, W1COLS), np.float32)
_SM1 = np.zeros((GW1, W1COLS), np.float32)
_SM1[16:, :] = _EYE128
_RM1 = np.zeros((GW1, W1COLS), np.float32)
for _h in range(HEADS):
  _RM1[_h, _h * HID:(_h + 1) * HID] = 1.0
_SM2 = np.zeros((GW2, OUT), np.float32)
_SM2[16:, :] = _EYE64
_RM2 = np.zeros((GW2, OUT), np.float32)
_RM2[0, :] = 1.0
_ZROWS1 = np.zeros((N // 16, GW1), np.float32)
_ZROWS2 = np.zeros((N // 16, GW2), np.float32)


def _comm_gather_kernel(n_pad):
  per_w = n_pad // NWORKERS
  steps = per_w // KCH
  mesh = plsc.VectorSubcoreMesh(core_axis_name="c", subcore_axis_name="s")

  @functools.partial(
      pl.kernel,
      mesh=mesh,
      out_type=jax.ShapeDtypeStruct((n_pad, 16), jnp.float32),
      compiler_params=pltpu.CompilerParams(use_tc_tiling_on_sc=False),
      scratch_types=[
          pltpu.VMEM((steps, KCH), jnp.int32),
          pltpu.VMEM((KCH, 16), jnp.float32),
          pltpu.SemaphoreType.DMA,
      ],
  )
  def k(ids_hbm, ctab_hbm, out_hbm, idxb, rows, sem):
    cid = lax.axis_index("c")
    sid = lax.axis_index("s")
    wid = cid * 16 + sid
    pltpu.sync_copy(ids_hbm.at[pl.ds(wid * steps, steps)], idxb)
    for kk in range(steps):
      pltpu.async_copy(ctab_hbm.at[idxb.at[kk]], rows, sem).wait()
      pltpu.sync_copy(rows, out_hbm.at[pl.ds(wid * per_w + kk * KCH, KCH)])

  return k


def _lane_bcast(v, lane):
  idx = jnp.full((16, 1), lane, jnp.int32)
  return lax.gather(
      v, idx,
      lax.GatherDimensionNumbers(
          offset_dims=(), collapsed_slice_dims=(0,), start_index_map=(0,)),
      (1,),
      mode=lax.GatherScatterMode.PROMISE_IN_BOUNDS)


def _edge_kernel(width, heads, kch):
  rows_per_tile = N // 16
  nchunk = EPW // kch
  nblock = nchunk // IB
  mesh = plsc.VectorSubcoreMesh(core_axis_name="c", subcore_axis_name="s")

  @functools.partial(
      pl.kernel,
      mesh=mesh,
      out_type=jax.ShapeDtypeStruct((2, N, width), jnp.float32),
      compiler_params=pltpu.CompilerParams(use_tc_tiling_on_sc=False),
      scratch_types=[
          pltpu.VMEM_SHARED((N, width), jnp.float32),
          pltpu.VMEM((IB, kch), jnp.int32),
          pltpu.VMEM((IB, kch), jnp.int32),
          pltpu.VMEM((kch, width), jnp.float32),
          pltpu.VMEM((kch, width), jnp.float32),
          pltpu.VMEM((kch, 16), jnp.float32),
          pltpu.VMEM((kch, width), jnp.float32),
          pltpu.VMEM((kch, width), jnp.float32),
          pltpu.SemaphoreType.DMA,
          pltpu.SemaphoreType.DMA,
          pltpu.SemaphoreType.DMA,
          pltpu.SemaphoreType.DMA,
          pltpu.SemaphoreType.DMA,
      ],
  )
  def k(g_hbm, d_hbm, src_hbm, dst_hbm, z_hbm, out_hbm,
        acc, sidx, didx, grow0, grow1, drow, msg0, msg1,
        sg0, sg1, sd, ssc0, ssc1):
    cid = lax.axis_index("c")
    sid = lax.axis_index("s")
    wid = cid * 16 + sid

    pltpu.sync_copy(z_hbm, acc.at[pl.ds(sid * rows_per_tile, rows_per_tile)])
    plsc.subcore_barrier()

    base = wid * nchunk
    nvec = width // 16

    def compute_chunk(gbuf, mbuf):
      @plsc.parallel_loop(0, kch)
      def edge_body(e):
        z = gbuf[e, pl.ds(0, 16)] + drow[e, pl.ds(0, 16)]
        z = jnp.where(z >= 0.0, z, z * jnp.float32(0.2))
        w = jnp.exp(z)
        mbuf[e, pl.ds(0, 16)] = w
        for kk in range(1, nvec):
          lane = (kk - 1) if heads == HEADS else 0
          wb = _lane_bcast(w, lane)
          mbuf[e, pl.ds(16 * kk, 16)] = gbuf[e, pl.ds(16 * kk, 16)] * wb

    def fire_g(j, gbuf, sem):
      pltpu.async_copy(g_hbm.at[sidx.at[j]], gbuf, sem)

    def wait_g(j, gbuf, sem):
      pltpu.make_async_copy(g_hbm.at[sidx.at[j]], gbuf, sem).wait()

    def fire_d(j):
      pltpu.async_copy(d_hbm.at[didx.at[j]], drow, sd)

    def wait_d(j):
      pltpu.make_async_copy(d_hbm.at[didx.at[j]], drow, sd).wait()

    def fire_sct(j, mbuf, sem):
      pltpu.async_copy(mbuf, acc.at[didx.at[j]], sem, add=True)

    def wait_sct(j, mbuf, sem):
      pltpu.make_async_copy(mbuf, acc.at[didx.at[j]], sem).wait()

    def block_body(jb, carry):
      pltpu.sync_copy(src_hbm.at[pl.ds(base + jb * IB, IB)], sidx)
      pltpu.sync_copy(dst_hbm.at[pl.ds(base + jb * IB, IB)], didx)
      fire_g(0, grow0, sg0)
      fire_d(0)
      wait_g(0, grow0, sg0)
      fire_g(1, grow1, sg1)
      wait_d(0)
      compute_chunk(grow0, msg0)
      fire_d(1)
      fire_sct(0, msg0, ssc0)
      wait_g(1, grow1, sg1)
      fire_g(2, grow0, sg0)
      wait_d(1)
      compute_chunk(grow1, msg1)
      fire_d(2)
      fire_sct(1, msg1, ssc1)

      def pair_body(jj, c):
        j0 = 2 * jj + 2
        wait_g(j0, grow0, sg0)
        fire_g(j0 + 1, grow1, sg1)
        wait_sct(j0 - 2, msg0, ssc0)
        wait_d(j0)
        compute_chunk(grow0, msg0)
        fire_d(j0 + 1)
        fire_sct(j0, msg0, ssc0)
        wait_g(j0 + 1, grow1, sg1)
        fire_g(j0 + 2, grow0, sg0)
        wait_sct(j0 - 1, msg1, ssc1)
        wait_d(j0 + 1)
        compute_chunk(grow1, msg1)
        fire_d(j0 + 2)
        fire_sct(j0 + 1, msg1, ssc1)
        return c

      lax.fori_loop(0, (IB - 3) // 2, pair_body, 0)
      wait_g(IB - 1, grow0, sg0)
      wait_sct(IB - 3, msg0, ssc0)
      wait_d(IB - 1)
      compute_chunk(grow0, msg0)
      fire_sct(IB - 1, msg0, ssc0)
      wait_sct(IB - 1, msg0, ssc0)
      wait_sct(IB - 2, msg1, ssc1)
      return carry

    lax.fori_loop(0, nblock, block_body, 0)
    plsc.subcore_barrier()
    pltpu.sync_copy(
        acc.at[pl.ds(sid * rows_per_tile, rows_per_tile)],
        out_hbm.at[cid, pl.ds(sid * rows_per_tile, rows_per_tile)])

  return k


def _tc1_kernel(x_blk, c_blk, w1x, w1c, p1, pd1, g_out, d_out):
  h1 = jnp.dot(x_blk[...], w1x[...], precision=lax.Precision.HIGHEST)
  h1 = h1 + jnp.dot(c_blk[...], w1c[...], precision=lax.Precision.HIGHEST)
  g_out[...] = jnp.dot(h1, p1[...], precision=lax.Precision.HIGHEST)
  d_out[...] = jnp.dot(h1, pd1[...], precision=lax.Precision.HIGHEST)


def _tc2_kernel(p_blk, sm, rm, b1_blk, w2, p2, pd2, g_out, d_out):
  accs = p_blk[0] + p_blk[1]
  den = jnp.dot(accs, rm[...], precision=lax.Precision.HIGHEST)
  msgs = jnp.dot(accs, sm[...], precision=lax.Precision.HIGHEST)
  h_mid = jnp.maximum(msgs / (den + 1e-16) + b1_blk[...], 0.0)
  h2 = jnp.dot(h_mid, w2[...], precision=lax.Precision.HIGHEST)
  g_out[...] = jnp.dot(h2, p2[...], precision=lax.Precision.HIGHEST)
  d_out[...] = jnp.dot(h2, pd2[...], precision=lax.Precision.HIGHEST)


def _tc3_kernel(p_blk, sm, rm, b2_blk, out_ref):
  accs = p_blk[0] + p_blk[1]
  den = jnp.dot(accs, rm[...], precision=lax.Precision.HIGHEST)
  msgs = jnp.dot(accs, sm[...], precision=lax.Precision.HIGHEST)
  out_ref[...] = msgs / (den + 1e-16) + b2_blk[...]


def kernel(x, edge_index, comm_ids, comm_table, W1, a_src1, a_dst1, b1,
           W2, a_src2, a_dst2, b2):
  f32 = jnp.float32

  w1x = W1[:D, :]
  w1c = jnp.concatenate([W1[D:, :], _Z8x128], axis=0)

  p1 = jnp.concatenate([a_src1.reshape(-1)[:, None] * _MASK1, _EYE128], axis=1)
  pd1 = a_dst1.reshape(-1)[:, None] * _MASK1

  sm1 = _SM1
  rm1 = _RM1

  p2 = jnp.concatenate([a_src2[0][:, None] * _MASK2, _EYE64], axis=1)
  pd2 = a_dst2[0][:, None] * _MASK2
  sm2 = _SM2
  rm2 = _RM2

  src1_2d = edge_index[0].reshape(E // K1, K1)
  dst1_2d = edge_index[1].reshape(E // K1, K1)
  src2_2d = edge_index[0].reshape(E // K2, K2)
  dst2_2d = edge_index[1].reshape(E // K2, K2)

  n_pad = 10240
  ids_pad = jnp.zeros((n_pad,), jnp.int32).at[:N].set(comm_ids)
  ids2d = ids_pad.reshape(n_pad // KCH, KCH)
  ctab_pad = jnp.zeros((NC, 16), f32).at[:, :CD].set(comm_table)

  z1 = jnp.asarray(_ZROWS1)
  z2 = jnp.asarray(_ZROWS2)

  c_full = _comm_gather_kernel(n_pad)(ids2d, ctab_pad)
  c_nodes = c_full[:N]

  nb = 5
  rb = N // nb
  g1, d1 = pl.pallas_call(
      _tc1_kernel,
      grid=(nb,),
      in_specs=[
          pl.BlockSpec((rb, D), lambda i: (i, 0)),
          pl.BlockSpec((rb, 16), lambda i: (i, 0)),
          pl.BlockSpec((D, W1COLS), lambda i: (0, 0)),
          pl.BlockSpec((16, W1COLS), lambda i: (0, 0)),
          pl.BlockSpec((W1COLS, GW1), lambda i: (0, 0)),
          pl.BlockSpec((W1COLS, 16), lambda i: (0, 0)),
      ],
      out_specs=[
          pl.BlockSpec((rb, GW1), lambda i: (i, 0)),
          pl.BlockSpec((rb, 16), lambda i: (i, 0)),
      ],
      out_shape=[
          jax.ShapeDtypeStruct((N, GW1), f32),
          jax.ShapeDtypeStruct((N, 16), f32),
      ],
  )(x, c_nodes, w1x, w1c, p1, pd1)

  part1 = _edge_kernel(GW1, HEADS, K1)(g1, d1, src1_2d, dst1_2d, z1)

  g2, d2 = pl.pallas_call(
      _tc2_kernel,
      grid=(nb,),
      in_specs=[
          pl.BlockSpec((2, rb, GW1), lambda i: (0, i, 0)),
          pl.BlockSpec((GW1, W1COLS), lambda i: (0, 0)),
          pl.BlockSpec((GW1, W1COLS), lambda i: (0, 0)),
          pl.BlockSpec((1, W1COLS), lambda i: (0, 0)),
          pl.BlockSpec((W1COLS, OUT), lambda i: (0, 0)),
          pl.BlockSpec((OUT, GW2), lambda i: (0, 0)),
          pl.BlockSpec((OUT, 16), lambda i: (0, 0)),
      ],
      out_specs=[
          pl.BlockSpec((rb, GW2), lambda i: (i, 0)),
          pl.BlockSpec((rb, 16), lambda i: (i, 0)),
      ],
      out_shape=[
          jax.ShapeDtypeStruct((N, GW2), f32),
          jax.ShapeDtypeStruct((N, 16), f32),
      ],
  )(part1, sm1, rm1, b1.reshape(1, W1COLS), W2, p2, pd2)

  part2 = _edge_kernel(GW2, 1, K2)(g2, d2, src2_2d, dst2_2d, z2)

  out = pl.pallas_call(
      _tc3_kernel,
      grid=(nb,),
      in_specs=[
          pl.BlockSpec((2, rb, GW2), lambda i: (0, i, 0)),
          pl.BlockSpec((GW2, OUT), lambda i: (0, 0)),
          pl.BlockSpec((GW2, OUT), lambda i: (0, 0)),
          pl.BlockSpec((1, OUT), lambda i: (0, 0)),
      ],
      out_specs=pl.BlockSpec((rb, OUT), lambda i: (i, 0)),
      out_shape=jax.ShapeDtypeStruct((N, OUT), f32),
  )(part2, sm2, rm2, b2.reshape(1, OUT))

  return out


# speedup vs baseline: 2.6296x; 1.0936x over previous
import functools

import jax
import jax.numpy as jnp
import numpy as np
from jax import lax
from jax.experimental import pallas as pl
from jax.experimental.pallas import tpu as pltpu
from jax.experimental.pallas import tpu_sc as plsc

N = 10000
E = 320000
D = 128
NC = 1000
CD = 8
HEADS = 8
HID = 16
OUT = 64

NWORKERS = 32
KCH = 80
K1 = 50
K2 = 80
EPW = E // NWORKERS
NCHUNK = EPW // KCH
IB = 25
W1COLS = HEADS * HID
GW1 = 16 + W1COLS
GW2 = 16 + OUT

_MASK1 = np.repeat(np.eye(HEADS, 16, dtype=np.float32), HID, axis=0)
_MASK2 = np.zeros((OUT, 16), np.float32)
_MASK2[:, 0] = 1.0
_EYE128 = np.eye(W1COLS, dtype=np.float32)
_EYE64 = np.eye(OUT, dtype=np.float32)
_Z8x128 = np.zeros((16 - CD, W1COLS), np.float32)
_SM1 = np.zeros((GW1, W1COLS), np.float32)
_SM1[16:, :] = _EYE128
_RM1 = np.zeros((GW1, W1COLS), np.float32)
for _h in range(HEADS):
  _RM1[_h, _h * HID:(_h + 1) * HID] = 1.0
_SM2 = np.zeros((GW2, OUT), np.float32)
_SM2[16:, :] = _EYE64
_RM2 = np.zeros((GW2, OUT), np.float32)
_RM2[0, :] = 1.0
_ZROWS1 = np.zeros((N // 16, GW1), np.float32)
_ZROWS2 = np.zeros((N // 16, GW2), np.float32)


def _comm_gather_kernel(n_pad):
  per_w = n_pad // NWORKERS
  steps = per_w // KCH
  mesh = plsc.VectorSubcoreMesh(core_axis_name="c", subcore_axis_name="s")

  @functools.partial(
      pl.kernel,
      mesh=mesh,
      out_type=jax.ShapeDtypeStruct((n_pad, 16), jnp.float32),
      compiler_params=pltpu.CompilerParams(use_tc_tiling_on_sc=False),
      scratch_types=[
          pltpu.VMEM((steps, KCH), jnp.int32),
          pltpu.VMEM((KCH, 16), jnp.float32),
          pltpu.SemaphoreType.DMA,
      ],
  )
  def k(ids_hbm, ctab_hbm, out_hbm, idxb, rows, sem):
    cid = lax.axis_index("c")
    sid = lax.axis_index("s")
    wid = cid * 16 + sid
    pltpu.sync_copy(ids_hbm.at[pl.ds(wid * steps, steps)], idxb)
    for kk in range(steps):
      pltpu.async_copy(ctab_hbm.at[idxb.at[kk]], rows, sem).wait()
      pltpu.sync_copy(rows, out_hbm.at[pl.ds(wid * per_w + kk * KCH, KCH)])

  return k


def _lane_bcast(v, lane):
  idx = jnp.full((16, 1), lane, jnp.int32)
  return lax.gather(
      v, idx,
      lax.GatherDimensionNumbers(
          offset_dims=(), collapsed_slice_dims=(0,), start_index_map=(0,)),
      (1,),
      mode=lax.GatherScatterMode.PROMISE_IN_BOUNDS)


def _edge_kernel(width, heads, kch):
  rows_per_tile = N // 16
  nchunk = EPW // kch
  nblock = nchunk // IB
  mesh = plsc.VectorSubcoreMesh(core_axis_name="c", subcore_axis_name="s")

  @functools.partial(
      pl.kernel,
      mesh=mesh,
      out_type=jax.ShapeDtypeStruct((2, N, width), jnp.float32),
      compiler_params=pltpu.CompilerParams(use_tc_tiling_on_sc=False),
      scratch_types=[
          pltpu.VMEM_SHARED((N, width), jnp.float32),
          pltpu.VMEM((IB, kch), jnp.int32),
          pltpu.VMEM((IB, kch), jnp.int32),
          pltpu.VMEM((kch, width), jnp.float32),
          pltpu.VMEM((kch, width), jnp.float32),
          pltpu.VMEM((kch, 16), jnp.float32),
          pltpu.VMEM((kch, width), jnp.float32),
          pltpu.VMEM((kch, width), jnp.float32),
          pltpu.SemaphoreType.DMA,
          pltpu.SemaphoreType.DMA,
          pltpu.SemaphoreType.DMA,
          pltpu.SemaphoreType.DMA,
          pltpu.SemaphoreType.DMA,
      ],
  )
  def k(g_hbm, d_hbm, src_hbm, dst_hbm, z_hbm, out_hbm,
        acc, sidx, didx, grow0, grow1, drow, msg0, msg1,
        sg0, sg1, sd, ssc0, ssc1):
    cid = lax.axis_index("c")
    sid = lax.axis_index("s")
    wid = cid * 16 + sid

    pltpu.sync_copy(z_hbm, acc.at[pl.ds(sid * rows_per_tile, rows_per_tile)])
    plsc.subcore_barrier()

    base = wid * nchunk
    nvec = width // 16

    def compute_chunk(gbuf, mbuf):
      @plsc.parallel_loop(0, kch)
      def edge_body(e):
        z = gbuf[e, pl.ds(0, 16)] + drow[e, pl.ds(0, 16)]
        z = jnp.where(z >= 0.0, z, z * jnp.float32(0.2))
        w = jnp.exp(z)
        mbuf[e, pl.ds(0, 16)] = w
        for kk in range(1, nvec):
          lane = (kk - 1) if heads == HEADS else 0
          wb = _lane_bcast(w, lane)
          mbuf[e, pl.ds(16 * kk, 16)] = gbuf[e, pl.ds(16 * kk, 16)] * wb

    def fire_g(j, gbuf, sem):
      pltpu.async_copy(g_hbm.at[sidx.at[j]], gbuf, sem)

    def wait_g(j, gbuf, sem):
      pltpu.make_async_copy(g_hbm.at[sidx.at[j]], gbuf, sem).wait()

    def fire_d(j):
      pltpu.async_copy(d_hbm.at[didx.at[j]], drow, sd)

    def wait_d(j):
      pltpu.make_async_copy(d_hbm.at[didx.at[j]], drow, sd).wait()

    def fire_sct(j, mbuf, sem):
      pltpu.async_copy(mbuf, acc.at[didx.at[j]], sem, add=True)

    def wait_sct(j, mbuf, sem):
      pltpu.make_async_copy(mbuf, acc.at[didx.at[j]], sem).wait()

    def block_body(jb, carry):
      pltpu.sync_copy(src_hbm.at[pl.ds(base + jb * IB, IB)], sidx)
      pltpu.sync_copy(dst_hbm.at[pl.ds(base + jb * IB, IB)], didx)
      fire_g(0, grow0, sg0)
      fire_d(0)
      wait_g(0, grow0, sg0)
      fire_g(1, grow1, sg1)
      wait_d(0)
      compute_chunk(grow0, msg0)
      fire_d(1)
      fire_sct(0, msg0, ssc0)
      wait_g(1, grow1, sg1)
      fire_g(2, grow0, sg0)
      wait_d(1)
      compute_chunk(grow1, msg1)
      fire_d(2)
      fire_sct(1, msg1, ssc1)

      def pair_body(jj, c):
        j0 = 2 * jj + 2
        wait_g(j0, grow0, sg0)
        fire_g(j0 + 1, grow1, sg1)
        wait_sct(j0 - 2, msg0, ssc0)
        wait_d(j0)
        compute_chunk(grow0, msg0)
        fire_d(j0 + 1)
        fire_sct(j0, msg0, ssc0)
        wait_g(j0 + 1, grow1, sg1)
        fire_g(j0 + 2, grow0, sg0)
        wait_sct(j0 - 1, msg1, ssc1)
        wait_d(j0 + 1)
        compute_chunk(grow1, msg1)
        fire_d(j0 + 2)
        fire_sct(j0 + 1, msg1, ssc1)
        return c

      lax.fori_loop(0, (IB - 3) // 2, pair_body, 0)
      wait_g(IB - 1, grow0, sg0)
      wait_sct(IB - 3, msg0, ssc0)
      wait_d(IB - 1)
      compute_chunk(grow0, msg0)
      fire_sct(IB - 1, msg0, ssc0)
      wait_sct(IB - 1, msg0, ssc0)
      wait_sct(IB - 2, msg1, ssc1)
      return carry

    lax.fori_loop(0, nblock, block_body, 0)
    plsc.subcore_barrier()
    pltpu.sync_copy(
        acc.at[pl.ds(sid * rows_per_tile, rows_per_tile)],
        out_hbm.at[cid, pl.ds(sid * rows_per_tile, rows_per_tile)])

  return k


def _tc1_kernel(x_blk, c_blk, w1x, w1c, p1, pd1, g_out, d_out):
  h1 = jnp.dot(x_blk[...], w1x[...], precision=lax.Precision.DEFAULT)
  h1 = h1 + jnp.dot(c_blk[...], w1c[...], precision=lax.Precision.DEFAULT)
  g_out[...] = jnp.dot(h1, p1[...], precision=lax.Precision.DEFAULT)
  d_out[...] = jnp.dot(h1, pd1[...], precision=lax.Precision.DEFAULT)


def _tc2_kernel(p_blk, sm, rm, b1_blk, w2, p2, pd2, g_out, d_out):
  accs = p_blk[0] + p_blk[1]
  den = jnp.dot(accs, rm[...], precision=lax.Precision.DEFAULT)
  msgs = jnp.dot(accs, sm[...], precision=lax.Precision.DEFAULT)
  h_mid = jnp.maximum(msgs / (den + 1e-16) + b1_blk[...], 0.0)
  h2 = jnp.dot(h_mid, w2[...], precision=lax.Precision.DEFAULT)
  g_out[...] = jnp.dot(h2, p2[...], precision=lax.Precision.DEFAULT)
  d_out[...] = jnp.dot(h2, pd2[...], precision=lax.Precision.DEFAULT)


def _tc3_kernel(p_blk, sm, rm, b2_blk, out_ref):
  accs = p_blk[0] + p_blk[1]
  den = jnp.dot(accs, rm[...], precision=lax.Precision.DEFAULT)
  msgs = jnp.dot(accs, sm[...], precision=lax.Precision.DEFAULT)
  out_ref[...] = msgs / (den + 1e-16) + b2_blk[...]


def kernel(x, edge_index, comm_ids, comm_table, W1, a_src1, a_dst1, b1,
           W2, a_src2, a_dst2, b2):
  f32 = jnp.float32

  w1x = W1[:D, :]
  w1c = jnp.concatenate([W1[D:, :], _Z8x128], axis=0)

  p1 = jnp.concatenate([a_src1.reshape(-1)[:, None] * _MASK1, _EYE128], axis=1)
  pd1 = a_dst1.reshape(-1)[:, None] * _MASK1

  sm1 = _SM1
  rm1 = _RM1

  p2 = jnp.concatenate([a_src2[0][:, None] * _MASK2, _EYE64], axis=1)
  pd2 = a_dst2[0][:, None] * _MASK2
  sm2 = _SM2
  rm2 = _RM2

  src1_2d = edge_index[0].reshape(E // K1, K1)
  dst1_2d = edge_index[1].reshape(E // K1, K1)
  src2_2d = edge_index[0].reshape(E // K2, K2)
  dst2_2d = edge_index[1].reshape(E // K2, K2)

  n_pad = 10240
  ids_pad = jnp.zeros((n_pad,), jnp.int32).at[:N].set(comm_ids)
  ids2d = ids_pad.reshape(n_pad // KCH, KCH)
  ctab_pad = jnp.zeros((NC, 16), f32).at[:, :CD].set(comm_table)

  z1 = jnp.asarray(_ZROWS1)
  z2 = jnp.asarray(_ZROWS2)

  c_full = _comm_gather_kernel(n_pad)(ids2d, ctab_pad)
  c_nodes = c_full[:N]

  nb = 5
  rb = N // nb
  g1, d1 = pl.pallas_call(
      _tc1_kernel,
      grid=(nb,),
      in_specs=[
          pl.BlockSpec((rb, D), lambda i: (i, 0)),
          pl.BlockSpec((rb, 16), lambda i: (i, 0)),
          pl.BlockSpec((D, W1COLS), lambda i: (0, 0)),
          pl.BlockSpec((16, W1COLS), lambda i: (0, 0)),
          pl.BlockSpec((W1COLS, GW1), lambda i: (0, 0)),
          pl.BlockSpec((W1COLS, 16), lambda i: (0, 0)),
      ],
      out_specs=[
          pl.BlockSpec((rb, GW1), lambda i: (i, 0)),
          pl.BlockSpec((rb, 16), lambda i: (i, 0)),
      ],
      out_shape=[
          jax.ShapeDtypeStruct((N, GW1), f32),
          jax.ShapeDtypeStruct((N, 16), f32),
      ],
  )(x, c_nodes, w1x, w1c, p1, pd1)

  part1 = _edge_kernel(GW1, HEADS, K1)(g1, d1, src1_2d, dst1_2d, z1)

  g2, d2 = pl.pallas_call(
      _tc2_kernel,
      grid=(nb,),
      in_specs=[
          pl.BlockSpec((2, rb, GW1), lambda i: (0, i, 0)),
          pl.BlockSpec((GW1, W1COLS), lambda i: (0, 0)),
          pl.BlockSpec((GW1, W1COLS), lambda i: (0, 0)),
          pl.BlockSpec((1, W1COLS), lambda i: (0, 0)),
          pl.BlockSpec((W1COLS, OUT), lambda i: (0, 0)),
          pl.BlockSpec((OUT, GW2), lambda i: (0, 0)),
          pl.BlockSpec((OUT, 16), lambda i: (0, 0)),
      ],
      out_specs=[
          pl.BlockSpec((rb, GW2), lambda i: (i, 0)),
          pl.BlockSpec((rb, 16), lambda i: (i, 0)),
      ],
      out_shape=[
          jax.ShapeDtypeStruct((N, GW2), f32),
          jax.ShapeDtypeStruct((N, 16), f32),
      ],
  )(part1, sm1, rm1, b1.reshape(1, W1COLS), W2, p2, pd2)

  part2 = _edge_kernel(GW2, 1, K2)(g2, d2, src2_2d, dst2_2d, z2)

  out = pl.pallas_call(
      _tc3_kernel,
      grid=(nb,),
      in_specs=[
          pl.BlockSpec((2, rb, GW2), lambda i: (0, i, 0)),
          pl.BlockSpec((GW2, OUT), lambda i: (0, 0)),
          pl.BlockSpec((GW2, OUT), lambda i: (0, 0)),
          pl.BlockSpec((1, OUT), lambda i: (0, 0)),
      ],
      out_specs=pl.BlockSpec((rb, OUT), lambda i: (i, 0)),
      out_shape=jax.ShapeDtypeStruct((N, OUT), f32),
  )(part2, sm2, rm2, b2.reshape(1, OUT))

  return out
